# Initial kernel scaffold; baseline (speedup 1.0000x reference)
import jax
import jax.numpy as jnp
from jax.experimental import pallas as pl


def kernel(x, edge_index, W1, b1, W2, b2):
    raise NotImplementedError("write your pallas kernel here")


# speedup vs baseline: 11.5895x; 11.5895x over previous
import functools

import jax
import jax.numpy as jnp
from jax import lax
from jax.experimental import pallas as pl
from jax.experimental.pallas import tpu as pltpu
from jax.experimental.pallas import tpu_sc as plsc

_NC = 2
_NS = 16
_K = 80
_DW = 16
_FB = 80


def _mesh():
  return plsc.VectorSubcoreMesh(core_axis_name="c", subcore_axis_name="s")


@functools.lru_cache(maxsize=None)
def _degree_kernel(N, E):
  NW = _NC * _NS
  EW = E // NW
  NCH = EW // _K
  TB = N // _FB
  BPT = -(-TB // _NS)

  @functools.partial(
      pl.kernel,
      out_type=jax.ShapeDtypeStruct((_NC, N, _DW), jnp.float32),
      mesh=_mesh(),
      scratch_types=[
          pltpu.VMEM((_K,), jnp.int32),
          pltpu.VMEM((_K, _DW), jnp.float32),
          pltpu.VMEM((_FB, _DW), jnp.float32),
          pltpu.VMEM_SHARED((N, _DW), jnp.float32),
      ],
      compiler_params=pltpu.CompilerParams(use_tc_tiling_on_sc=False),
  )
  def deg_k(dst_hbm, out_hbm, dst_v, ones_v, buf_v, acc_sh):
    c = lax.axis_index("c")
    s = lax.axis_index("s")
    wid = c * _NS + s
    one = jnp.full((16,), 1.0, jnp.float32)
    zero = jnp.zeros((16,), jnp.float32)

    def fill(i, carry):
      ones_v[i, :] = one
      return carry

    lax.fori_loop(0, _K, fill, 0)

    def zrow(i, carry):
      buf_v[i, :] = zero
      return carry

    lax.fori_loop(0, _FB, zrow, 0)
    for j in range(BPT):
      blk = s + j * _NS

      @pl.when(blk < TB)
      def _():
        pltpu.sync_copy(buf_v, acc_sh.at[pl.ds(blk * _FB, _FB), :])

    plsc.subcore_barrier()

    def body(i, carry):
      pltpu.sync_copy(dst_hbm.at[pl.ds(wid * EW + i * _K, _K)], dst_v)
      pltpu.sync_copy(ones_v, acc_sh.at[dst_v], add=True)
      return carry

    lax.fori_loop(0, NCH, body, 0)
    plsc.subcore_barrier()
    for j in range(BPT):
      blk = s + j * _NS

      @pl.when(blk < TB)
      def _():
        r = blk * _FB
        pltpu.sync_copy(acc_sh.at[pl.ds(r, _FB), :], buf_v)
        pltpu.sync_copy(buf_v, out_hbm.at[c, pl.ds(r, _FB), :])

  return deg_k


@functools.lru_cache(maxsize=None)
def _spmm_kernel(N, E, W):
  NW = _NC * _NS
  EW = E // NW
  NCH = EW // _K
  TB = N // _FB
  BPT = -(-TB // _NS)

  @functools.partial(
      pl.kernel,
      out_type=jax.ShapeDtypeStruct((_NC, N, W), jnp.float32),
      mesh=_mesh(),
      scratch_types=[
          pltpu.VMEM((_K,), jnp.int32),
          pltpu.VMEM((_K,), jnp.int32),
          pltpu.VMEM((_K, W), jnp.float32),
          pltpu.VMEM((_FB, W), jnp.float32),
          pltpu.VMEM_SHARED((N, W), jnp.float32),
      ],
      compiler_params=pltpu.CompilerParams(use_tc_tiling_on_sc=False),
  )
  def spmm_k(t_hbm, src_hbm, dst_hbm, out_hbm, src_v, dst_v, rows_v, buf_v,
             acc_sh):
    c = lax.axis_index("c")
    s = lax.axis_index("s")
    wid = c * _NS + s
    zero = jnp.zeros((16,), jnp.float32)

    def zrow(i, carry):
      for j in range(W // 16):
        buf_v[i, pl.ds(j * 16, 16)] = zero
      return carry

    lax.fori_loop(0, _FB, zrow, 0)
    for j in range(BPT):
      blk = s + j * _NS

      @pl.when(blk < TB)
      def _():
        pltpu.sync_copy(buf_v, acc_sh.at[pl.ds(blk * _FB, _FB), :])

    plsc.subcore_barrier()

    def body(i, carry):
      off = wid * EW + i * _K
      pltpu.sync_copy(src_hbm.at[pl.ds(off, _K)], src_v)
      pltpu.sync_copy(dst_hbm.at[pl.ds(off, _K)], dst_v)
      pltpu.sync_copy(t_hbm.at[src_v], rows_v)
      pltpu.sync_copy(rows_v, acc_sh.at[dst_v], add=True)
      return carry

    lax.fori_loop(0, NCH, body, 0)
    plsc.subcore_barrier()
    for j in range(BPT):
      blk = s + j * _NS

      @pl.when(blk < TB)
      def _():
        r = blk * _FB
        pltpu.sync_copy(acc_sh.at[pl.ds(r, _FB), :], buf_v)
        pltpu.sync_copy(buf_v, out_hbm.at[c, pl.ds(r, _FB), :])

  return spmm_k


def _tc1(deg_parts, x, W1, R=1000):
  N, NF = x.shape
  NH = W1.shape[1]

  def body(dp_ref, x_ref, w_ref, t_ref, dis_ref):
    deg = dp_ref[0][:, 0:1] + dp_ref[1][:, 0:1]
    dis = lax.rsqrt(jnp.maximum(deg, 1.0))
    s = jnp.dot(x_ref[...], w_ref[...], preferred_element_type=jnp.float32)
    t_ref[...] = s * dis
    dis_ref[...] = dis

  return pl.pallas_call(
      body,
      grid=(N // R,),
      in_specs=[
          pl.BlockSpec((_NC, R, _DW), lambda i: (0, i, 0)),
          pl.BlockSpec((R, NF), lambda i: (i, 0)),
          pl.BlockSpec((NF, NH), lambda i: (0, 0)),
      ],
      out_specs=[
          pl.BlockSpec((R, NH), lambda i: (i, 0)),
          pl.BlockSpec((R, 1), lambda i: (i, 0)),
      ],
      out_shape=[
          jax.ShapeDtypeStruct((N, NH), jnp.float32),
          jax.ShapeDtypeStruct((N, 1), jnp.float32),
      ],
  )(deg_parts, x, W1)


def _tc2(p1, dis, b1, W2p, R=1000):
  _, N, NH = p1.shape
  WP = W2p.shape[1]

  def body(p_ref, dis_ref, b_ref, w_ref, t_ref):
    d = dis_ref[...]
    h = (p_ref[0] + p_ref[1]) * d + b_ref[...]
    h = jnp.maximum(h, 0.0)
    t_ref[...] = jnp.dot(h, w_ref[...],
                         preferred_element_type=jnp.float32) * d

  return pl.pallas_call(
      body,
      grid=(N // R,),
      in_specs=[
          pl.BlockSpec((_NC, R, NH), lambda i: (0, i, 0)),
          pl.BlockSpec((R, 1), lambda i: (i, 0)),
          pl.BlockSpec((1, NH), lambda i: (0, 0)),
          pl.BlockSpec((NH, WP), lambda i: (0, 0)),
      ],
      out_specs=pl.BlockSpec((R, WP), lambda i: (i, 0)),
      out_shape=jax.ShapeDtypeStruct((N, WP), jnp.float32),
  )(p1, dis, b1, W2p)


def _tc3(p2, dis, b2, R=1000):
  _, N, WP = p2.shape
  C = b2.shape[1]

  def body(p_ref, dis_ref, b_ref, o_ref):
    d = dis_ref[...]
    h = (p_ref[0] + p_ref[1])[:, :C] * d + b_ref[...]
    m = jnp.max(h, axis=1, keepdims=True)
    lse = m + jnp.log(jnp.sum(jnp.exp(h - m), axis=1, keepdims=True))
    o_ref[...] = h - lse

  return pl.pallas_call(
      body,
      grid=(N // R,),
      in_specs=[
          pl.BlockSpec((_NC, R, WP), lambda i: (0, i, 0)),
          pl.BlockSpec((R, 1), lambda i: (i, 0)),
          pl.BlockSpec((1, C), lambda i: (0, 0)),
      ],
      out_specs=pl.BlockSpec((R, C), lambda i: (i, 0)),
      out_shape=jax.ShapeDtypeStruct((N, C), jnp.float32),
  )(p2, dis, b2)


def kernel(x, edge_index, W1, b1, W2, b2):
  N, _ = x.shape
  NH = W1.shape[1]
  C = W2.shape[1]
  E = edge_index.shape[1]
  src = edge_index[0]
  dst = edge_index[1]
  WP = 48
  W2p = jnp.pad(W2, ((0, 0), (0, WP - C)))

  deg_parts = _degree_kernel(N, E)(dst)
  t1, dis = _tc1(deg_parts, x, W1)
  p1 = _spmm_kernel(N, E, NH)(t1, src, dst)
  t2 = _tc2(p1, dis, b1.reshape(1, -1), W2p)
  p2 = _spmm_kernel(N, E, WP)(t2, src, dst)
  return _tc3(p2, dis, b2.reshape(1, -1))


# speedup vs baseline: 24.2161x; 2.0895x over previous
import functools

import jax
import jax.numpy as jnp
from jax import lax
from jax.experimental import pallas as pl
from jax.experimental.pallas import tpu as pltpu
from jax.experimental.pallas import tpu_sc as plsc

_NC = 2
_NS = 16
_K = 80
_DW = 16
_FB = 80


def _mesh():
  return plsc.VectorSubcoreMesh(core_axis_name="c", subcore_axis_name="s")


@functools.lru_cache(maxsize=None)
def _degree_kernel(N, E):
  NW = _NC * _NS
  EW = E // NW
  K2 = 125
  NCH = EW // K2
  G = 8
  TB = N // _FB
  BPT = -(-TB // _NS)

  @functools.partial(
      pl.kernel,
      out_type=jax.ShapeDtypeStruct((_NC, N, _DW), jnp.float32),
      mesh=_mesh(),
      scratch_types=[
          pltpu.VMEM((NCH, K2), jnp.int32),
          pltpu.VMEM((K2, _DW), jnp.float32),
          pltpu.VMEM((_FB, _DW), jnp.float32),
          pltpu.VMEM_SHARED((N, _DW), jnp.float32),
          pltpu.SemaphoreType.DMA,
          pltpu.SemaphoreType.DMA,
      ],
      compiler_params=pltpu.CompilerParams(use_tc_tiling_on_sc=False),
  )
  def deg_k(dst_hbm, out_hbm, dst_all, ones_v, buf_v, acc_sh, isem, ssem):
    c = lax.axis_index("c")
    s = lax.axis_index("s")
    wid = c * _NS + s
    one = jnp.full((16,), 1.0, jnp.float32)
    zero = jnp.zeros((16,), jnp.float32)

    pltpu.async_copy(dst_hbm.at[wid], dst_all, isem)

    def fill(i, carry):
      ones_v[i, :] = one
      return carry

    lax.fori_loop(0, K2, fill, 0)

    def zrow(i, carry):
      buf_v[i, :] = zero
      return carry

    lax.fori_loop(0, _FB, zrow, 0)
    for j in range(BPT):
      blk = s + j * _NS

      @pl.when(blk < TB)
      def _():
        pltpu.sync_copy(buf_v, acc_sh.at[pl.ds(blk * _FB, _FB), :])

    pltpu.make_async_copy(dst_hbm.at[wid], dst_all, isem).wait()
    plsc.subcore_barrier()

    def body(g, carry):
      for j in range(G):
        pltpu.async_copy(ones_v, acc_sh.at[dst_all.at[g * G + j]], ssem,
                         add=True)
      for j in range(G):
        pltpu.make_async_copy(ones_v, acc_sh.at[dst_all.at[0]], ssem).wait()
      return carry

    lax.fori_loop(0, NCH // G, body, 0)
    plsc.subcore_barrier()
    for j in range(BPT):
      blk = s + j * _NS

      @pl.when(blk < TB)
      def _():
        r = blk * _FB
        pltpu.sync_copy(acc_sh.at[pl.ds(r, _FB), :], buf_v)
        pltpu.sync_copy(buf_v, out_hbm.at[c, pl.ds(r, _FB), :])

  return deg_k


@functools.lru_cache(maxsize=None)
def _spmm_kernel(N, E, W):
  NW = _NC * _NS
  EW = E // NW
  K2 = 100
  NCH = EW // K2
  NP = NCH // 2
  TB = N // _FB
  BPT = -(-TB // _NS)

  @functools.partial(
      pl.kernel,
      out_type=jax.ShapeDtypeStruct((_NC, N, W), jnp.float32),
      mesh=_mesh(),
      scratch_types=[
          pltpu.VMEM((NCH, K2), jnp.int32),
          pltpu.VMEM((NCH, K2), jnp.int32),
          pltpu.VMEM((K2, W), jnp.float32),
          pltpu.VMEM((K2, W), jnp.float32),
          pltpu.VMEM_SHARED((N, W), jnp.float32),
          pltpu.SemaphoreType.DMA,
          pltpu.SemaphoreType.DMA,
          pltpu.SemaphoreType.DMA,
          pltpu.SemaphoreType.DMA,
      ],
      compiler_params=pltpu.CompilerParams(use_tc_tiling_on_sc=False),
  )
  def spmm_k(t_hbm, src_hbm, dst_hbm, out_hbm, src_all, dst_all, rows0, rows1,
             acc_sh, gs0, gs1, ss0, ss1):
    c = lax.axis_index("c")
    s = lax.axis_index("s")
    wid = c * _NS + s
    zero = jnp.zeros((16,), jnp.float32)

    pltpu.async_copy(src_hbm.at[wid], src_all, gs0)
    pltpu.async_copy(dst_hbm.at[wid], dst_all, gs1)

    def zrow(i, carry):
      for j in range(W // 16):
        rows0[i, pl.ds(j * 16, 16)] = zero
      return carry

    lax.fori_loop(0, _FB, zrow, 0)
    for j in range(BPT):
      blk = s + j * _NS

      @pl.when(blk < TB)
      def _():
        pltpu.sync_copy(rows0.at[pl.ds(0, _FB), :],
                        acc_sh.at[pl.ds(blk * _FB, _FB), :])

    pltpu.make_async_copy(src_hbm.at[wid], src_all, gs0).wait()
    pltpu.make_async_copy(dst_hbm.at[wid], dst_all, gs1).wait()
    plsc.subcore_barrier()

    pltpu.async_copy(t_hbm.at[src_all.at[0]], rows0, gs0)

    def body(p, carry):
      i0 = 2 * p
      i1 = i0 + 1
      pltpu.make_async_copy(t_hbm.at[src_all.at[i0]], rows0, gs0).wait()

      @pl.when(p > 0)
      def _():
        pltpu.make_async_copy(rows1, acc_sh.at[dst_all.at[i1]], ss1).wait()

      pltpu.async_copy(rows0, acc_sh.at[dst_all.at[i0]], ss0, add=True)
      pltpu.async_copy(t_hbm.at[src_all.at[i1]], rows1, gs1)
      pltpu.make_async_copy(t_hbm.at[src_all.at[i1]], rows1, gs1).wait()
      pltpu.make_async_copy(rows0, acc_sh.at[dst_all.at[i0]], ss0).wait()
      pltpu.async_copy(rows1, acc_sh.at[dst_all.at[i1]], ss1, add=True)

      @pl.when(p < NP - 1)
      def _():
        pltpu.async_copy(t_hbm.at[src_all.at[i0 + 2]], rows0, gs0)

      return carry

    lax.fori_loop(0, NP, body, 0)
    pltpu.make_async_copy(rows1, acc_sh.at[dst_all.at[NCH - 1]], ss1).wait()
    plsc.subcore_barrier()
    for j in range(BPT):
      blk = s + j * _NS

      @pl.when(blk < TB)
      def _():
        r = blk * _FB
        pltpu.sync_copy(acc_sh.at[pl.ds(r, _FB), :], rows0.at[pl.ds(0, _FB), :])
        pltpu.sync_copy(rows0.at[pl.ds(0, _FB), :],
                        out_hbm.at[c, pl.ds(r, _FB), :])

  return spmm_k


def _tc1(deg_parts, x, W1, R=1000):
  N, NF = x.shape
  NH = W1.shape[1]

  def body(dp_ref, x_ref, w_ref, t_ref, dis_ref):
    deg = dp_ref[0][:, 0:1] + dp_ref[1][:, 0:1]
    dis = lax.rsqrt(jnp.maximum(deg, 1.0))
    s = jnp.dot(x_ref[...], w_ref[...], preferred_element_type=jnp.float32)
    t_ref[...] = s * dis
    dis_ref[...] = dis

  return pl.pallas_call(
      body,
      grid=(N // R,),
      in_specs=[
          pl.BlockSpec((_NC, R, _DW), lambda i: (0, i, 0)),
          pl.BlockSpec((R, NF), lambda i: (i, 0)),
          pl.BlockSpec((NF, NH), lambda i: (0, 0)),
      ],
      out_specs=[
          pl.BlockSpec((R, NH), lambda i: (i, 0)),
          pl.BlockSpec((R, 1), lambda i: (i, 0)),
      ],
      out_shape=[
          jax.ShapeDtypeStruct((N, NH), jnp.float32),
          jax.ShapeDtypeStruct((N, 1), jnp.float32),
      ],
  )(deg_parts, x, W1)


def _tc2(p1, dis, b1, W2p, R=1000):
  _, N, NH = p1.shape
  WP = W2p.shape[1]

  def body(p_ref, dis_ref, b_ref, w_ref, t_ref):
    d = dis_ref[...]
    h = (p_ref[0] + p_ref[1]) * d + b_ref[...]
    h = jnp.maximum(h, 0.0)
    t_ref[...] = jnp.dot(h, w_ref[...],
                         preferred_element_type=jnp.float32) * d

  return pl.pallas_call(
      body,
      grid=(N // R,),
      in_specs=[
          pl.BlockSpec((_NC, R, NH), lambda i: (0, i, 0)),
          pl.BlockSpec((R, 1), lambda i: (i, 0)),
          pl.BlockSpec((1, NH), lambda i: (0, 0)),
          pl.BlockSpec((NH, WP), lambda i: (0, 0)),
      ],
      out_specs=pl.BlockSpec((R, WP), lambda i: (i, 0)),
      out_shape=jax.ShapeDtypeStruct((N, WP), jnp.float32),
  )(p1, dis, b1, W2p)


def _tc3(p2, dis, b2, R=1000):
  _, N, WP = p2.shape
  C = b2.shape[1]

  def body(p_ref, dis_ref, b_ref, o_ref):
    d = dis_ref[...]
    h = (p_ref[0] + p_ref[1])[:, :C] * d + b_ref[...]
    m = jnp.max(h, axis=1, keepdims=True)
    lse = m + jnp.log(jnp.sum(jnp.exp(h - m), axis=1, keepdims=True))
    o_ref[...] = h - lse

  return pl.pallas_call(
      body,
      grid=(N // R,),
      in_specs=[
          pl.BlockSpec((_NC, R, WP), lambda i: (0, i, 0)),
          pl.BlockSpec((R, 1), lambda i: (i, 0)),
          pl.BlockSpec((1, C), lambda i: (0, 0)),
      ],
      out_specs=pl.BlockSpec((R, C), lambda i: (i, 0)),
      out_shape=jax.ShapeDtypeStruct((N, C), jnp.float32),
  )(p2, dis, b2)


def kernel(x, edge_index, W1, b1, W2, b2):
  N, _ = x.shape
  NH = W1.shape[1]
  C = W2.shape[1]
  E = edge_index.shape[1]
  src = edge_index[0]
  dst = edge_index[1]
  WP = 48
  W2p = jnp.pad(W2, ((0, 0), (0, WP - C)))
  NW = _NC * _NS
  src_r = src.reshape(NW, -1, 100)
  dst_r = dst.reshape(NW, -1, 100)

  deg_parts = _degree_kernel(N, E)(dst.reshape(NW, -1, 125))
  t1, dis = _tc1(deg_parts, x, W1)
  p1 = _spmm_kernel(N, E, NH)(t1, src_r, dst_r)
  t2 = _tc2(p1, dis, b1.reshape(1, -1), W2p)
  p2 = _spmm_kernel(N, E, WP)(t2, src_r, dst_r)
  return _tc3(p2, dis, b2.reshape(1, -1))


# speedup vs baseline: 24.7579x; 1.0224x over previous
import functools

import jax
import jax.numpy as jnp
from jax import lax
from jax.experimental import pallas as pl
from jax.experimental.pallas import tpu as pltpu
from jax.experimental.pallas import tpu_sc as plsc

_NC = 2
_NS = 16
_K = 80
_DW = 16
_FB = 80


def _mesh():
  return plsc.VectorSubcoreMesh(core_axis_name="c", subcore_axis_name="s")


@functools.lru_cache(maxsize=None)
def _degree_kernel(N, E):
  NW = _NC * _NS
  EW = E // NW
  K2 = 125
  NCH = EW // K2
  G = 8
  TB = N // _FB
  BPT = -(-TB // _NS)

  @functools.partial(
      pl.kernel,
      out_type=jax.ShapeDtypeStruct((_NC, N, _DW), jnp.float32),
      mesh=_mesh(),
      scratch_types=[
          pltpu.VMEM((NCH, K2), jnp.int32),
          pltpu.VMEM((K2, _DW), jnp.float32),
          pltpu.VMEM((_FB, _DW), jnp.float32),
          pltpu.VMEM_SHARED((N, _DW), jnp.float32),
          pltpu.SemaphoreType.DMA,
          pltpu.SemaphoreType.DMA,
      ],
      compiler_params=pltpu.CompilerParams(use_tc_tiling_on_sc=False),
  )
  def deg_k(dst_hbm, out_hbm, dst_all, ones_v, buf_v, acc_sh, isem, ssem):
    c = lax.axis_index("c")
    s = lax.axis_index("s")
    wid = c * _NS + s
    one = jnp.full((16,), 1.0, jnp.float32)
    zero = jnp.zeros((16,), jnp.float32)

    pltpu.async_copy(dst_hbm.at[wid], dst_all, isem)

    def fill(i, carry):
      ones_v[i, :] = one
      return carry

    lax.fori_loop(0, K2, fill, 0)

    def zrow(i, carry):
      buf_v[i, :] = zero
      return carry

    lax.fori_loop(0, _FB, zrow, 0)
    for j in range(BPT):
      blk = s + j * _NS

      @pl.when(blk < TB)
      def _():
        pltpu.sync_copy(buf_v, acc_sh.at[pl.ds(blk * _FB, _FB), :])

    pltpu.make_async_copy(dst_hbm.at[wid], dst_all, isem).wait()
    plsc.subcore_barrier()

    def body(g, carry):
      for j in range(G):
        pltpu.async_copy(ones_v, acc_sh.at[dst_all.at[g * G + j]], ssem,
                         add=True)
      for j in range(G):
        pltpu.make_async_copy(ones_v, acc_sh.at[dst_all.at[0]], ssem).wait()
      return carry

    lax.fori_loop(0, NCH // G, body, 0)
    plsc.subcore_barrier()
    for j in range(BPT):
      blk = s + j * _NS

      @pl.when(blk < TB)
      def _():
        r = blk * _FB
        pltpu.sync_copy(acc_sh.at[pl.ds(r, _FB), :], buf_v)
        pltpu.sync_copy(buf_v, out_hbm.at[c, pl.ds(r, _FB), :])

  return deg_k


@functools.lru_cache(maxsize=None)
def _spmm_kernel(N, E, W):
  NW = _NC * _NS
  EW = E // NW
  K2 = 100 if W > 64 else 125
  NCH = EW // K2
  NP = NCH // 2
  TB = N // _FB
  BPT = -(-TB // _NS)

  @functools.partial(
      pl.kernel,
      out_type=jax.ShapeDtypeStruct((_NC, N, W), jnp.float32),
      mesh=_mesh(),
      scratch_types=[
          pltpu.VMEM((NCH, K2), jnp.int32),
          pltpu.VMEM((NCH, K2), jnp.int32),
          pltpu.VMEM((K2, W), jnp.float32),
          pltpu.VMEM((K2, W), jnp.float32),
          pltpu.VMEM_SHARED((N, W), jnp.float32),
          pltpu.SemaphoreType.DMA,
          pltpu.SemaphoreType.DMA,
          pltpu.SemaphoreType.DMA,
          pltpu.SemaphoreType.DMA,
      ],
      compiler_params=pltpu.CompilerParams(use_tc_tiling_on_sc=False),
  )
  def spmm_k(t_hbm, src_hbm, dst_hbm, out_hbm, src_all, dst_all, rows0, rows1,
             acc_sh, gs0, gs1, ss0, ss1):
    c = lax.axis_index("c")
    s = lax.axis_index("s")
    wid = c * _NS + s
    zero = jnp.zeros((16,), jnp.float32)

    pltpu.async_copy(src_hbm.at[wid], src_all, gs0)
    pltpu.async_copy(dst_hbm.at[wid], dst_all, gs1)

    def zrow(i, carry):
      for j in range(W // 16):
        rows0[i, pl.ds(j * 16, 16)] = zero
      return carry

    lax.fori_loop(0, _FB, zrow, 0)
    for j in range(BPT):
      blk = s + j * _NS

      @pl.when(blk < TB)
      def _():
        pltpu.sync_copy(rows0.at[pl.ds(0, _FB), :],
                        acc_sh.at[pl.ds(blk * _FB, _FB), :])

    pltpu.make_async_copy(src_hbm.at[wid], src_all, gs0).wait()
    pltpu.make_async_copy(dst_hbm.at[wid], dst_all, gs1).wait()
    plsc.subcore_barrier()

    pltpu.async_copy(t_hbm.at[src_all.at[0]], rows0, gs0)

    def body(p, carry):
      i0 = 2 * p
      i1 = i0 + 1
      pltpu.make_async_copy(t_hbm.at[src_all.at[i0]], rows0, gs0).wait()

      @pl.when(p > 0)
      def _():
        pltpu.make_async_copy(rows1, acc_sh.at[dst_all.at[i1]], ss1).wait()

      pltpu.async_copy(rows0, acc_sh.at[dst_all.at[i0]], ss0, add=True)
      pltpu.async_copy(t_hbm.at[src_all.at[i1]], rows1, gs1)
      pltpu.make_async_copy(t_hbm.at[src_all.at[i1]], rows1, gs1).wait()
      pltpu.make_async_copy(rows0, acc_sh.at[dst_all.at[i0]], ss0).wait()
      pltpu.async_copy(rows1, acc_sh.at[dst_all.at[i1]], ss1, add=True)

      @pl.when(p < NP - 1)
      def _():
        pltpu.async_copy(t_hbm.at[src_all.at[i0 + 2]], rows0, gs0)

      return carry

    lax.fori_loop(0, NP, body, 0)
    pltpu.make_async_copy(rows1, acc_sh.at[dst_all.at[NCH - 1]], ss1).wait()
    plsc.subcore_barrier()
    for j in range(BPT):
      blk = s + j * _NS

      @pl.when(blk < TB)
      def _():
        r = blk * _FB
        pltpu.sync_copy(acc_sh.at[pl.ds(r, _FB), :], rows0.at[pl.ds(0, _FB), :])
        pltpu.sync_copy(rows0.at[pl.ds(0, _FB), :],
                        out_hbm.at[c, pl.ds(r, _FB), :])

  return spmm_k


def _tc_mm(x, W1, R=1000):
  N, NF = x.shape
  NH = W1.shape[1]

  def body(x_ref, w_ref, s_ref):
    s_ref[...] = jnp.dot(x_ref[...], w_ref[...],
                         preferred_element_type=jnp.float32)

  return pl.pallas_call(
      body,
      grid=(N // R,),
      in_specs=[
          pl.BlockSpec((R, NF), lambda i: (i, 0)),
          pl.BlockSpec((NF, NH), lambda i: (0, 0)),
      ],
      out_specs=pl.BlockSpec((R, NH), lambda i: (i, 0)),
      out_shape=jax.ShapeDtypeStruct((N, NH), jnp.float32),
  )(x, W1)


def _tc_scale(deg_parts, s1, R=1000):
  N, NH = s1.shape

  def body(dp_ref, s_ref, t_ref, dis_ref):
    deg = dp_ref[0][:, 0:1] + dp_ref[1][:, 0:1]
    dis = lax.rsqrt(jnp.maximum(deg, 1.0))
    t_ref[...] = s_ref[...] * dis
    dis_ref[...] = dis

  return pl.pallas_call(
      body,
      grid=(N // R,),
      in_specs=[
          pl.BlockSpec((_NC, R, _DW), lambda i: (0, i, 0)),
          pl.BlockSpec((R, NH), lambda i: (i, 0)),
      ],
      out_specs=[
          pl.BlockSpec((R, NH), lambda i: (i, 0)),
          pl.BlockSpec((R, 1), lambda i: (i, 0)),
      ],
      out_shape=[
          jax.ShapeDtypeStruct((N, NH), jnp.float32),
          jax.ShapeDtypeStruct((N, 1), jnp.float32),
      ],
  )(deg_parts, s1)


def _tc2(p1, dis, b1, W2p, R=1000):
  _, N, NH = p1.shape
  WP = W2p.shape[1]

  def body(p_ref, dis_ref, b_ref, w_ref, t_ref):
    d = dis_ref[...]
    h = (p_ref[0] + p_ref[1]) * d + b_ref[...]
    h = jnp.maximum(h, 0.0)
    t_ref[...] = jnp.dot(h, w_ref[...],
                         preferred_element_type=jnp.float32) * d

  return pl.pallas_call(
      body,
      grid=(N // R,),
      in_specs=[
          pl.BlockSpec((_NC, R, NH), lambda i: (0, i, 0)),
          pl.BlockSpec((R, 1), lambda i: (i, 0)),
          pl.BlockSpec((1, NH), lambda i: (0, 0)),
          pl.BlockSpec((NH, WP), lambda i: (0, 0)),
      ],
      out_specs=pl.BlockSpec((R, WP), lambda i: (i, 0)),
      out_shape=jax.ShapeDtypeStruct((N, WP), jnp.float32),
  )(p1, dis, b1, W2p)


def _tc3(p2, dis, b2, R=1000):
  _, N, WP = p2.shape
  C = b2.shape[1]

  def body(p_ref, dis_ref, b_ref, o_ref):
    d = dis_ref[...]
    h = (p_ref[0] + p_ref[1])[:, :C] * d + b_ref[...]
    m = jnp.max(h, axis=1, keepdims=True)
    lse = m + jnp.log(jnp.sum(jnp.exp(h - m), axis=1, keepdims=True))
    o_ref[...] = h - lse

  return pl.pallas_call(
      body,
      grid=(N // R,),
      in_specs=[
          pl.BlockSpec((_NC, R, WP), lambda i: (0, i, 0)),
          pl.BlockSpec((R, 1), lambda i: (i, 0)),
          pl.BlockSpec((1, C), lambda i: (0, 0)),
      ],
      out_specs=pl.BlockSpec((R, C), lambda i: (i, 0)),
      out_shape=jax.ShapeDtypeStruct((N, C), jnp.float32),
  )(p2, dis, b2)


def kernel(x, edge_index, W1, b1, W2, b2):
  N, _ = x.shape
  NH = W1.shape[1]
  C = W2.shape[1]
  E = edge_index.shape[1]
  src = edge_index[0]
  dst = edge_index[1]
  WP = 48
  W2p = jnp.pad(W2, ((0, 0), (0, WP - C)))
  NW = _NC * _NS
  src_r = src.reshape(NW, -1, 100)
  dst_r = dst.reshape(NW, -1, 100)
  src_r2 = src.reshape(NW, -1, 125)
  dst_r2 = dst.reshape(NW, -1, 125)

  deg_parts = _degree_kernel(N, E)(dst.reshape(NW, -1, 125))
  s1 = _tc_mm(x, W1)
  t1, dis = _tc_scale(deg_parts, s1)
  p1 = _spmm_kernel(N, E, NH)(t1, src_r, dst_r)
  t2 = _tc2(p1, dis, b1.reshape(1, -1), W2p)
  p2 = _spmm_kernel(N, E, WP)(t2, src_r2, dst_r2)
  return _tc3(p2, dis, b2.reshape(1, -1))


# speedup vs baseline: 26.8840x; 1.0859x over previous
import functools

import jax
import jax.numpy as jnp
from jax import lax
from jax.experimental import pallas as pl
from jax.experimental.pallas import tpu as pltpu
from jax.experimental.pallas import tpu_sc as plsc

_NC = 2
_NS = 16
_K = 80
_DW = 16
_FB = 80


def _mesh():
  return plsc.VectorSubcoreMesh(core_axis_name="c", subcore_axis_name="s")


@functools.lru_cache(maxsize=None)
def _degree_kernel(N, E):
  NW = _NC * _NS
  EW = E // NW
  K2 = 125
  NCH = EW // K2
  G = 8
  TB = N // _FB
  BPT = -(-TB // _NS)

  @functools.partial(
      pl.kernel,
      out_type=jax.ShapeDtypeStruct((_NC, N, _DW), jnp.float32),
      mesh=_mesh(),
      scratch_types=[
          pltpu.VMEM((NCH, K2), jnp.int32),
          pltpu.VMEM((K2, _DW), jnp.float32),
          pltpu.VMEM((_FB, _DW), jnp.float32),
          pltpu.VMEM_SHARED((N, _DW), jnp.float32),
          pltpu.SemaphoreType.DMA,
          pltpu.SemaphoreType.DMA,
      ],
      compiler_params=pltpu.CompilerParams(use_tc_tiling_on_sc=False),
  )
  def deg_k(dst_hbm, out_hbm, dst_all, ones_v, buf_v, acc_sh, isem, ssem):
    c = lax.axis_index("c")
    s = lax.axis_index("s")
    wid = c * _NS + s
    one = jnp.full((16,), 1.0, jnp.float32)
    zero = jnp.zeros((16,), jnp.float32)

    pltpu.async_copy(dst_hbm.at[wid], dst_all, isem)

    def fill(i, carry):
      ones_v[i, :] = one
      return carry

    lax.fori_loop(0, K2, fill, 0)

    def zrow(i, carry):
      buf_v[i, :] = zero
      return carry

    lax.fori_loop(0, _FB, zrow, 0)
    for j in range(BPT):
      blk = s + j * _NS

      @pl.when(blk < TB)
      def _():
        pltpu.sync_copy(buf_v, acc_sh.at[pl.ds(blk * _FB, _FB), :])

    pltpu.make_async_copy(dst_hbm.at[wid], dst_all, isem).wait()
    plsc.subcore_barrier()

    def body(g, carry):
      for j in range(G):
        pltpu.async_copy(ones_v, acc_sh.at[dst_all.at[g * G + j]], ssem,
                         add=True)
      for j in range(G):
        pltpu.make_async_copy(ones_v, acc_sh.at[dst_all.at[0]], ssem).wait()
      return carry

    lax.fori_loop(0, NCH // G, body, 0)
    plsc.subcore_barrier()
    for j in range(BPT):
      blk = s + j * _NS

      @pl.when(blk < TB)
      def _():
        r = blk * _FB
        pltpu.sync_copy(acc_sh.at[pl.ds(r, _FB), :], buf_v)
        pltpu.sync_copy(buf_v, out_hbm.at[c, pl.ds(r, _FB), :])

  return deg_k


@functools.lru_cache(maxsize=None)
def _spmm_kernel(N, E, W):
  NW = _NC * _NS
  EW = E // NW
  K2 = 50 if W > 64 else 125
  NCH = EW // K2
  Q = NCH // 4
  FBk = 40 if W > 64 else 80
  TB = N // FBk
  BPT = -(-TB // _NS)

  @functools.partial(
      pl.kernel,
      out_type=jax.ShapeDtypeStruct((_NC, N, W), jnp.float32),
      mesh=_mesh(),
      scratch_types=[
          pltpu.VMEM((NCH, K2), jnp.int32),
          pltpu.VMEM((NCH, K2), jnp.int32),
          pltpu.VMEM((K2, W), jnp.float32),
          pltpu.VMEM((K2, W), jnp.float32),
          pltpu.VMEM((K2, W), jnp.float32),
          pltpu.VMEM((K2, W), jnp.float32),
          pltpu.VMEM_SHARED((N, W), jnp.float32),
          pltpu.SemaphoreType.DMA,
          pltpu.SemaphoreType.DMA,
          pltpu.SemaphoreType.DMA,
          pltpu.SemaphoreType.DMA,
          pltpu.SemaphoreType.DMA,
          pltpu.SemaphoreType.DMA,
          pltpu.SemaphoreType.DMA,
          pltpu.SemaphoreType.DMA,
      ],
      compiler_params=pltpu.CompilerParams(use_tc_tiling_on_sc=False),
  )
  def spmm_k(t_hbm, src_hbm, dst_hbm, out_hbm, src_all, dst_all,
             r0, r1, r2, r3, acc_sh, g0, g1, g2, g3, s0, s1, s2, s3):
    rows = (r0, r1, r2, r3)
    gs = (g0, g1, g2, g3)
    ss = (s0, s1, s2, s3)
    c = lax.axis_index("c")
    s = lax.axis_index("s")
    wid = c * _NS + s
    zero = jnp.zeros((16,), jnp.float32)

    pltpu.async_copy(src_hbm.at[wid], src_all, g0)
    pltpu.async_copy(dst_hbm.at[wid], dst_all, g1)

    def zrow(i, carry):
      for j in range(W // 16):
        r0[i, pl.ds(j * 16, 16)] = zero
      return carry

    lax.fori_loop(0, FBk, zrow, 0)
    for j in range(BPT):
      blk = s + j * _NS

      @pl.when(blk < TB)
      def _():
        pltpu.sync_copy(r0.at[pl.ds(0, FBk), :],
                        acc_sh.at[pl.ds(blk * FBk, FBk), :])

    pltpu.make_async_copy(src_hbm.at[wid], src_all, g0).wait()
    pltpu.make_async_copy(dst_hbm.at[wid], dst_all, g1).wait()
    plsc.subcore_barrier()

    pltpu.async_copy(t_hbm.at[src_all.at[0]], r0, g0)
    pltpu.async_copy(t_hbm.at[src_all.at[1]], r1, g1)

    def body(q, carry):
      for j in range(4):
        i = 4 * q + j
        bg = (j + 2) % 4
        pltpu.make_async_copy(t_hbm.at[src_all.at[i]], rows[j], gs[j]).wait()

        def wait_prev_scatter(i=i, bg=bg):
          pltpu.make_async_copy(rows[bg], acc_sh.at[dst_all.at[i]],
                                ss[bg]).wait()

        if j >= 2:
          wait_prev_scatter()
        else:
          pl.when(q > 0)(wait_prev_scatter)

        pltpu.async_copy(rows[j], acc_sh.at[dst_all.at[i]], ss[j], add=True)

        def start_next_gather(i=i, bg=bg):
          pltpu.async_copy(t_hbm.at[src_all.at[i + 2]], rows[bg], gs[bg])

        if j < 2:
          start_next_gather()
        else:
          pl.when(q < Q - 1)(start_next_gather)
      return carry

    lax.fori_loop(0, Q, body, 0)
    pltpu.make_async_copy(r2, acc_sh.at[dst_all.at[NCH - 2]], s2).wait()
    pltpu.make_async_copy(r3, acc_sh.at[dst_all.at[NCH - 1]], s3).wait()
    plsc.subcore_barrier()
    for j in range(BPT):
      blk = s + j * _NS

      @pl.when(blk < TB)
      def _():
        r = blk * FBk
        pltpu.sync_copy(acc_sh.at[pl.ds(r, FBk), :], r0.at[pl.ds(0, FBk), :])
        pltpu.sync_copy(r0.at[pl.ds(0, FBk), :],
                        out_hbm.at[c, pl.ds(r, FBk), :])

  return spmm_k


def _tc_mm(x, W1, R=1000):
  N, NF = x.shape
  NH = W1.shape[1]

  def body(x_ref, w_ref, s_ref):
    s_ref[...] = jnp.dot(x_ref[...], w_ref[...],
                         preferred_element_type=jnp.float32)

  return pl.pallas_call(
      body,
      grid=(N // R,),
      in_specs=[
          pl.BlockSpec((R, NF), lambda i: (i, 0)),
          pl.BlockSpec((NF, NH), lambda i: (0, 0)),
      ],
      out_specs=pl.BlockSpec((R, NH), lambda i: (i, 0)),
      out_shape=jax.ShapeDtypeStruct((N, NH), jnp.float32),
  )(x, W1)


def _tc_scale(deg_parts, s1, R=1000):
  N, NH = s1.shape

  def body(dp_ref, s_ref, t_ref, dis_ref):
    deg = dp_ref[0][:, 0:1] + dp_ref[1][:, 0:1]
    dis = lax.rsqrt(jnp.maximum(deg, 1.0))
    t_ref[...] = s_ref[...] * dis
    dis_ref[...] = dis

  return pl.pallas_call(
      body,
      grid=(N // R,),
      in_specs=[
          pl.BlockSpec((_NC, R, _DW), lambda i: (0, i, 0)),
          pl.BlockSpec((R, NH), lambda i: (i, 0)),
      ],
      out_specs=[
          pl.BlockSpec((R, NH), lambda i: (i, 0)),
          pl.BlockSpec((R, 1), lambda i: (i, 0)),
      ],
      out_shape=[
          jax.ShapeDtypeStruct((N, NH), jnp.float32),
          jax.ShapeDtypeStruct((N, 1), jnp.float32),
      ],
  )(deg_parts, s1)


def _tc2(p1, dis, b1, W2p, R=1000):
  _, N, NH = p1.shape
  WP = W2p.shape[1]

  def body(p_ref, dis_ref, b_ref, w_ref, t_ref):
    d = dis_ref[...]
    h = (p_ref[0] + p_ref[1]) * d + b_ref[...]
    h = jnp.maximum(h, 0.0)
    t_ref[...] = jnp.dot(h, w_ref[...],
                         preferred_element_type=jnp.float32) * d

  return pl.pallas_call(
      body,
      grid=(N // R,),
      in_specs=[
          pl.BlockSpec((_NC, R, NH), lambda i: (0, i, 0)),
          pl.BlockSpec((R, 1), lambda i: (i, 0)),
          pl.BlockSpec((1, NH), lambda i: (0, 0)),
          pl.BlockSpec((NH, WP), lambda i: (0, 0)),
      ],
      out_specs=pl.BlockSpec((R, WP), lambda i: (i, 0)),
      out_shape=jax.ShapeDtypeStruct((N, WP), jnp.float32),
  )(p1, dis, b1, W2p)


def _tc3(p2, dis, b2, R=1000):
  _, N, WP = p2.shape
  C = b2.shape[1]

  def body(p_ref, dis_ref, b_ref, o_ref):
    d = dis_ref[...]
    h = (p_ref[0] + p_ref[1])[:, :C] * d + b_ref[...]
    m = jnp.max(h, axis=1, keepdims=True)
    lse = m + jnp.log(jnp.sum(jnp.exp(h - m), axis=1, keepdims=True))
    o_ref[...] = h - lse

  return pl.pallas_call(
      body,
      grid=(N // R,),
      in_specs=[
          pl.BlockSpec((_NC, R, WP), lambda i: (0, i, 0)),
          pl.BlockSpec((R, 1), lambda i: (i, 0)),
          pl.BlockSpec((1, C), lambda i: (0, 0)),
      ],
      out_specs=pl.BlockSpec((R, C), lambda i: (i, 0)),
      out_shape=jax.ShapeDtypeStruct((N, C), jnp.float32),
  )(p2, dis, b2)


def kernel(x, edge_index, W1, b1, W2, b2):
  N, _ = x.shape
  NH = W1.shape[1]
  C = W2.shape[1]
  E = edge_index.shape[1]
  src = edge_index[0]
  dst = edge_index[1]
  WP = 48
  W2p = jnp.pad(W2, ((0, 0), (0, WP - C)))
  NW = _NC * _NS
  src_r = src.reshape(NW, -1, 50)
  dst_r = dst.reshape(NW, -1, 50)
  src_r2 = src.reshape(NW, -1, 125)
  dst_r2 = dst.reshape(NW, -1, 125)

  deg_parts = _degree_kernel(N, E)(dst.reshape(NW, -1, 125))
  s1 = _tc_mm(x, W1)
  t1, dis = _tc_scale(deg_parts, s1)
  p1 = _spmm_kernel(N, E, NH)(t1, src_r, dst_r)
  t2 = _tc2(p1, dis, b1.reshape(1, -1), W2p)
  p2 = _spmm_kernel(N, E, WP)(t2, src_r2, dst_r2)
  return _tc3(p2, dis, b2.reshape(1, -1))


# speedup vs baseline: 29.7968x; 1.1083x over previous
import functools

import jax
import jax.numpy as jnp
from jax import lax
from jax.experimental import pallas as pl
from jax.experimental.pallas import tpu as pltpu
from jax.experimental.pallas import tpu_sc as plsc

_NC = 2
_NS = 16
_K = 80
_DW = 16
_FB = 80


def _mesh():
  return plsc.VectorSubcoreMesh(core_axis_name="c", subcore_axis_name="s")


_K2 = 50


@functools.lru_cache(maxsize=None)
def _degree_kernel(N, E):
  NW = _NC * _NS
  EW = E // NW
  K2 = _K2
  NCH = EW // K2
  G = 8
  TB = N // _FB
  BPT = -(-TB // _NS)

  @functools.partial(
      pl.kernel,
      out_type=jax.ShapeDtypeStruct((_NC, N, _DW), jnp.float32),
      mesh=_mesh(),
      scratch_types=[
          pltpu.VMEM((NCH, K2), jnp.int32),
          pltpu.VMEM((K2, _DW), jnp.float32),
          pltpu.VMEM((_FB, _DW), jnp.float32),
          pltpu.VMEM_SHARED((N, _DW), jnp.float32),
          pltpu.SemaphoreType.DMA,
          pltpu.SemaphoreType.DMA,
      ],
      compiler_params=pltpu.CompilerParams(use_tc_tiling_on_sc=False),
  )
  def deg_k(ei_hbm, out_hbm, dst_all, ones_v, buf_v, acc_sh, isem, ssem):
    c = lax.axis_index("c")
    s = lax.axis_index("s")
    wid = c * _NS + s
    one = jnp.full((16,), 1.0, jnp.float32)
    zero = jnp.zeros((16,), jnp.float32)

    pltpu.async_copy(ei_hbm.at[1, wid], dst_all, isem)

    def fill(i, carry):
      ones_v[i, :] = one
      return carry

    lax.fori_loop(0, K2, fill, 0)

    def zrow(i, carry):
      buf_v[i, :] = zero
      return carry

    lax.fori_loop(0, _FB, zrow, 0)
    for j in range(BPT):
      blk = s + j * _NS

      @pl.when(blk < TB)
      def _():
        pltpu.sync_copy(buf_v, acc_sh.at[pl.ds(blk * _FB, _FB), :])

    pltpu.make_async_copy(ei_hbm.at[1, wid], dst_all, isem).wait()
    plsc.subcore_barrier()

    def body(g, carry):
      for j in range(G):
        pltpu.async_copy(ones_v, acc_sh.at[dst_all.at[g * G + j]], ssem,
                         add=True)
      for j in range(G):
        pltpu.make_async_copy(ones_v, acc_sh.at[dst_all.at[0]], ssem).wait()
      return carry

    lax.fori_loop(0, NCH // G, body, 0)
    plsc.subcore_barrier()
    for j in range(BPT):
      blk = s + j * _NS

      @pl.when(blk < TB)
      def _():
        r = blk * _FB
        pltpu.sync_copy(acc_sh.at[pl.ds(r, _FB), :], buf_v)
        pltpu.sync_copy(buf_v, out_hbm.at[c, pl.ds(r, _FB), :])

  return deg_k


@functools.lru_cache(maxsize=None)
def _spmm_kernel(N, E, W):
  NW = _NC * _NS
  EW = E // NW
  K2 = _K2
  NCH = EW // K2
  D = 5 if W > 64 else 8
  S = 3 if W > 64 else 5
  Q = NCH // D
  FBk = 40
  TB = N // FBk
  BPT = -(-TB // _NS)

  @functools.partial(
      pl.kernel,
      out_type=jax.ShapeDtypeStruct((_NC, N, W), jnp.float32),
      mesh=_mesh(),
      scratch_types=(
          [pltpu.VMEM((NCH, K2), jnp.int32)]
          + [pltpu.VMEM((K2,), jnp.int32) for _ in range(D)]
          + [pltpu.VMEM((K2, W), jnp.float32) for _ in range(D)]
          + [pltpu.VMEM_SHARED((N, W), jnp.float32)]
          + [pltpu.SemaphoreType.DMA for _ in range(3 * D + 1)]
      ),
      compiler_params=pltpu.CompilerParams(use_tc_tiling_on_sc=False),
  )
  def spmm_k(t_hbm, ei_hbm, out_hbm, *scr):
    dst_all = scr[0]
    srcb = scr[1:1 + D]
    rows = scr[1 + D:1 + 2 * D]
    acc_sh = scr[1 + 2 * D]
    gs = scr[2 + 2 * D:2 + 3 * D]
    ss = scr[2 + 3 * D:2 + 4 * D]
    isems = scr[2 + 4 * D:2 + 5 * D]
    dsem = scr[2 + 5 * D]
    c = lax.axis_index("c")
    s = lax.axis_index("s")
    wid = c * _NS + s
    zero = jnp.zeros((16,), jnp.float32)

    pltpu.async_copy(ei_hbm.at[1, wid], dst_all, dsem)

    r0 = rows[0]

    def zrow(i, carry):
      for j in range(W // 16):
        r0[i, pl.ds(j * 16, 16)] = zero
      return carry

    lax.fori_loop(0, FBk, zrow, 0)
    for j in range(BPT):
      blk = s + j * _NS

      @pl.when(blk < TB)
      def _():
        pltpu.sync_copy(r0.at[pl.ds(0, FBk), :],
                        acc_sh.at[pl.ds(blk * FBk, FBk), :])

    pltpu.make_async_copy(ei_hbm.at[1, wid], dst_all, dsem).wait()
    plsc.subcore_barrier()

    for k in range(S + 2):
      pltpu.async_copy(ei_hbm.at[0, wid, k], srcb[k % D], isems[k % D])
    for k in range(S):
      pltpu.make_async_copy(ei_hbm.at[0, wid, k], srcb[k % D],
                            isems[k % D]).wait()
      pltpu.async_copy(t_hbm.at[srcb[k % D]], rows[k % D], gs[k % D])

    def body(q, carry):
      for j in range(D):
        i = D * q + j
        bn = (j + S) % D
        bp = (j + S + 2) % D
        pltpu.make_async_copy(t_hbm.at[srcb[j]], rows[j], gs[j]).wait()

        def wait_prev_scatter(i=i, bn=bn):
          pltpu.make_async_copy(rows[bn], acc_sh.at[dst_all.at[i]],
                                ss[bn]).wait()

        if j >= D - S:
          wait_prev_scatter()
        else:
          pl.when(q > 0)(wait_prev_scatter)

        pltpu.async_copy(rows[j], acc_sh.at[dst_all.at[i]], ss[j], add=True)

        def start_next_gather(bn=bn):
          pltpu.make_async_copy(ei_hbm.at[0, wid, 0], srcb[bn],
                                isems[bn]).wait()
          pltpu.async_copy(t_hbm.at[srcb[bn]], rows[bn], gs[bn])

        if j < D - S:
          start_next_gather()
        else:
          pl.when(q < Q - 1)(start_next_gather)

        def prefetch_idx(i=i, bp=bp):
          pltpu.async_copy(ei_hbm.at[0, wid, i + S + 2], srcb[bp], isems[bp])

        if j < D - S - 2:
          prefetch_idx()
        else:
          pl.when(q < Q - 1)(prefetch_idx)
      return carry

    lax.fori_loop(0, Q, body, 0)
    for k in range(NCH - (D - S), NCH):
      pltpu.make_async_copy(rows[k % D], acc_sh.at[dst_all.at[NCH - 1]],
                            ss[k % D]).wait()
    plsc.subcore_barrier()
    for j in range(BPT):
      blk = s + j * _NS

      @pl.when(blk < TB)
      def _():
        r = blk * FBk
        pltpu.sync_copy(acc_sh.at[pl.ds(r, FBk), :], r0.at[pl.ds(0, FBk), :])
        pltpu.sync_copy(r0.at[pl.ds(0, FBk), :],
                        out_hbm.at[c, pl.ds(r, FBk), :])

  return spmm_k


def _tc1(deg_parts, x, W1, R=1000):
  N, NF = x.shape
  NH = W1.shape[1]

  def body(dp_ref, x_ref, w_ref, t_ref, dis_ref):
    deg = dp_ref[0][:, 0:1] + dp_ref[1][:, 0:1]
    dis = lax.rsqrt(jnp.maximum(deg, 1.0))
    s = jnp.dot(x_ref[...], w_ref[...], preferred_element_type=jnp.float32)
    t_ref[...] = s * dis
    dis_ref[...] = dis

  return pl.pallas_call(
      body,
      grid=(N // R,),
      in_specs=[
          pl.BlockSpec((_NC, R, _DW), lambda i: (0, i, 0)),
          pl.BlockSpec((R, NF), lambda i: (i, 0)),
          pl.BlockSpec((NF, NH), lambda i: (0, 0)),
      ],
      out_specs=[
          pl.BlockSpec((R, NH), lambda i: (i, 0)),
          pl.BlockSpec((R, 1), lambda i: (i, 0)),
      ],
      out_shape=[
          jax.ShapeDtypeStruct((N, NH), jnp.float32),
          jax.ShapeDtypeStruct((N, 1), jnp.float32),
      ],
  )(deg_parts, x, W1)


def _tc2(p1, dis, b1, W2p, R=1000):
  _, N, NH = p1.shape
  WP = W2p.shape[1]

  def body(p_ref, dis_ref, b_ref, w_ref, t_ref):
    d = dis_ref[...]
    h = (p_ref[0] + p_ref[1]) * d + b_ref[...]
    h = jnp.maximum(h, 0.0)
    t_ref[...] = jnp.dot(h, w_ref[...],
                         preferred_element_type=jnp.float32) * d

  return pl.pallas_call(
      body,
      grid=(N // R,),
      in_specs=[
          pl.BlockSpec((_NC, R, NH), lambda i: (0, i, 0)),
          pl.BlockSpec((R, 1), lambda i: (i, 0)),
          pl.BlockSpec((1, NH), lambda i: (0, 0)),
          pl.BlockSpec((NH, WP), lambda i: (0, 0)),
      ],
      out_specs=pl.BlockSpec((R, WP), lambda i: (i, 0)),
      out_shape=jax.ShapeDtypeStruct((N, WP), jnp.float32),
  )(p1, dis, b1, W2p)


def _tc3(p2, dis, b2, R=1000):
  _, N, WP = p2.shape
  C = b2.shape[1]

  def body(p_ref, dis_ref, b_ref, o_ref):
    d = dis_ref[...]
    h = (p_ref[0] + p_ref[1])[:, :C] * d + b_ref[...]
    m = jnp.max(h, axis=1, keepdims=True)
    lse = m + jnp.log(jnp.sum(jnp.exp(h - m), axis=1, keepdims=True))
    o_ref[...] = h - lse

  return pl.pallas_call(
      body,
      grid=(N // R,),
      in_specs=[
          pl.BlockSpec((_NC, R, WP), lambda i: (0, i, 0)),
          pl.BlockSpec((R, 1), lambda i: (i, 0)),
          pl.BlockSpec((1, C), lambda i: (0, 0)),
      ],
      out_specs=pl.BlockSpec((R, C), lambda i: (i, 0)),
      out_shape=jax.ShapeDtypeStruct((N, C), jnp.float32),
  )(p2, dis, b2)


def kernel(x, edge_index, W1, b1, W2, b2):
  N, _ = x.shape
  NH = W1.shape[1]
  C = W2.shape[1]
  E = edge_index.shape[1]
  WP = 48
  W2p = jnp.pad(W2, ((0, 0), (0, WP - C)))
  NW = _NC * _NS
  ei_r = edge_index.reshape(2, NW, -1, _K2)

  deg_parts = _degree_kernel(N, E)(ei_r)
  t1, dis = _tc1(deg_parts, x, W1)
  p1 = _spmm_kernel(N, E, NH)(t1, ei_r)
  t2 = _tc2(p1, dis, b1.reshape(1, -1), W2p)
  p2 = _spmm_kernel(N, E, WP)(t2, ei_r)
  return _tc3(p2, dis, b2.reshape(1, -1))


# speedup vs baseline: 30.5153x; 1.0241x over previous
import functools

import jax
import jax.numpy as jnp
from jax import lax
from jax.experimental import pallas as pl
from jax.experimental.pallas import tpu as pltpu
from jax.experimental.pallas import tpu_sc as plsc

_NC = 2
_NS = 16
_K = 80
_DW = 16
_FB = 80


def _mesh():
  return plsc.VectorSubcoreMesh(core_axis_name="c", subcore_axis_name="s")


_K2 = 50


@functools.lru_cache(maxsize=None)
def _degree_kernel(N, E):
  NW = _NC * _NS
  EW = E // NW
  K2 = _K2
  NCH = EW // K2
  G = 8
  TB = N // _FB
  BPT = -(-TB // _NS)

  @functools.partial(
      pl.kernel,
      out_type=jax.ShapeDtypeStruct((_NC, N, _DW), jnp.float32),
      mesh=_mesh(),
      scratch_types=[
          pltpu.VMEM((NCH, K2), jnp.int32),
          pltpu.VMEM((K2, _DW), jnp.float32),
          pltpu.VMEM((_FB, _DW), jnp.float32),
          pltpu.VMEM_SHARED((N, _DW), jnp.float32),
          pltpu.SemaphoreType.DMA,
          pltpu.SemaphoreType.DMA,
      ],
      compiler_params=pltpu.CompilerParams(use_tc_tiling_on_sc=False),
  )
  def deg_k(ei_hbm, out_hbm, dst_all, ones_v, buf_v, acc_sh, isem, ssem):
    c = lax.axis_index("c")
    s = lax.axis_index("s")
    wid = c * _NS + s
    one = jnp.full((16,), 1.0, jnp.float32)
    zero = jnp.zeros((16,), jnp.float32)

    pltpu.async_copy(ei_hbm.at[1, wid], dst_all, isem)

    def fill(i, carry):
      ones_v[i, :] = one
      return carry

    lax.fori_loop(0, K2, fill, 0)

    def zrow(i, carry):
      buf_v[i, :] = zero
      return carry

    lax.fori_loop(0, _FB, zrow, 0)
    for j in range(BPT):
      blk = s + j * _NS

      @pl.when(blk < TB)
      def _():
        pltpu.sync_copy(buf_v, acc_sh.at[pl.ds(blk * _FB, _FB), :])

    pltpu.make_async_copy(ei_hbm.at[1, wid], dst_all, isem).wait()
    plsc.subcore_barrier()

    def body(g, carry):
      for j in range(G):
        pltpu.async_copy(ones_v, acc_sh.at[dst_all.at[g * G + j]], ssem,
                         add=True)
      for j in range(G):
        pltpu.make_async_copy(ones_v, acc_sh.at[dst_all.at[0]], ssem).wait()
      return carry

    lax.fori_loop(0, NCH // G, body, 0)
    plsc.subcore_barrier()
    for j in range(BPT):
      blk = s + j * _NS

      @pl.when(blk < TB)
      def _():
        r = blk * _FB
        pltpu.sync_copy(acc_sh.at[pl.ds(r, _FB), :], buf_v)
        pltpu.sync_copy(buf_v, out_hbm.at[c, pl.ds(r, _FB), :])

  return deg_k


@functools.lru_cache(maxsize=None)
def _spmm_kernel(N, E, W):
  NW = _NC * _NS
  EW = E // NW
  K2 = _K2
  NCH = EW // K2
  D = 5 if W > 64 else 8
  S = 3 if W > 64 else 5
  Q = NCH // D
  FBk = 40
  TB = N // FBk
  BPT = -(-TB // _NS)

  @functools.partial(
      pl.kernel,
      out_type=jax.ShapeDtypeStruct((_NC, N, W), jnp.float32),
      mesh=_mesh(),
      scratch_types=(
          [pltpu.VMEM((NCH, K2), jnp.int32)]
          + [pltpu.VMEM((K2,), jnp.int32) for _ in range(D)]
          + [pltpu.VMEM((K2, W), jnp.float32) for _ in range(D)]
          + [pltpu.VMEM_SHARED((N, W), jnp.float32)]
          + [pltpu.SemaphoreType.DMA for _ in range(3 * D + 1)]
      ),
      compiler_params=pltpu.CompilerParams(use_tc_tiling_on_sc=False),
  )
  def spmm_k(t_hbm, ei_hbm, out_hbm, *scr):
    dst_all = scr[0]
    srcb = scr[1:1 + D]
    rows = scr[1 + D:1 + 2 * D]
    acc_sh = scr[1 + 2 * D]
    gs = scr[2 + 2 * D:2 + 3 * D]
    ss = scr[2 + 3 * D:2 + 4 * D]
    isems = scr[2 + 4 * D:2 + 5 * D]
    dsem = scr[2 + 5 * D]
    c = lax.axis_index("c")
    s = lax.axis_index("s")
    wid = c * _NS + s
    zero = jnp.zeros((16,), jnp.float32)

    pltpu.async_copy(ei_hbm.at[1, wid], dst_all, dsem)

    r0 = rows[0]

    def zrow(i, carry):
      for j in range(W // 16):
        r0[i, pl.ds(j * 16, 16)] = zero
      return carry

    lax.fori_loop(0, FBk, zrow, 0)
    for j in range(BPT):
      blk = s + j * _NS

      @pl.when(blk < TB)
      def _():
        pltpu.sync_copy(r0.at[pl.ds(0, FBk), :],
                        acc_sh.at[pl.ds(blk * FBk, FBk), :])

    pltpu.make_async_copy(ei_hbm.at[1, wid], dst_all, dsem).wait()
    plsc.subcore_barrier()

    for k in range(S + 2):
      pltpu.async_copy(ei_hbm.at[0, wid, k], srcb[k % D], isems[k % D])
    for k in range(S):
      pltpu.make_async_copy(ei_hbm.at[0, wid, k], srcb[k % D],
                            isems[k % D]).wait()
      pltpu.async_copy(t_hbm.at[srcb[k % D]], rows[k % D], gs[k % D])

    def body(q, carry):
      for j in range(D):
        i = D * q + j
        bn = (j + S) % D
        bp = (j + S + 2) % D
        pltpu.make_async_copy(t_hbm.at[srcb[j]], rows[j], gs[j]).wait()

        def wait_prev_scatter(i=i, bn=bn):
          pltpu.make_async_copy(rows[bn], acc_sh.at[dst_all.at[i]],
                                ss[bn]).wait()

        if j >= D - S:
          wait_prev_scatter()
        else:
          pl.when(q > 0)(wait_prev_scatter)

        pltpu.async_copy(rows[j], acc_sh.at[dst_all.at[i]], ss[j], add=True)

        def start_next_gather(bn=bn):
          pltpu.make_async_copy(ei_hbm.at[0, wid, 0], srcb[bn],
                                isems[bn]).wait()
          pltpu.async_copy(t_hbm.at[srcb[bn]], rows[bn], gs[bn])

        if j < D - S:
          start_next_gather()
        else:
          pl.when(q < Q - 1)(start_next_gather)

        def prefetch_idx(i=i, bp=bp):
          pltpu.async_copy(ei_hbm.at[0, wid, i + S + 2], srcb[bp], isems[bp])

        if j < D - S - 2:
          prefetch_idx()
        else:
          pl.when(q < Q - 1)(prefetch_idx)
      return carry

    lax.fori_loop(0, Q, body, 0)
    for k in range(NCH - (D - S), NCH):
      pltpu.make_async_copy(rows[k % D], acc_sh.at[dst_all.at[NCH - 1]],
                            ss[k % D]).wait()
    plsc.subcore_barrier()
    for j in range(BPT):
      blk = s + j * _NS

      @pl.when(blk < TB)
      def _():
        r = blk * FBk
        pltpu.sync_copy(acc_sh.at[pl.ds(r, FBk), :], r0.at[pl.ds(0, FBk), :])
        pltpu.sync_copy(r0.at[pl.ds(0, FBk), :],
                        out_hbm.at[c, pl.ds(r, FBk), :])

  return spmm_k


def _tc1(deg_parts, x, W1, R=2000):
  N, NF = x.shape
  NH = W1.shape[1]

  def body(dp_ref, x_ref, w_ref, t_ref, dis_ref):
    deg = dp_ref[0][:, 0:1] + dp_ref[1][:, 0:1]
    dis = lax.rsqrt(jnp.maximum(deg, 1.0))
    s = jnp.dot(x_ref[...], w_ref[...], preferred_element_type=jnp.float32)
    t_ref[...] = s * dis
    dis_ref[...] = dis

  return pl.pallas_call(
      body,
      grid=(N // R,),
      in_specs=[
          pl.BlockSpec((_NC, R, _DW), lambda i: (0, i, 0)),
          pl.BlockSpec((R, NF), lambda i: (i, 0)),
          pl.BlockSpec((NF, NH), lambda i: (0, 0)),
      ],
      out_specs=[
          pl.BlockSpec((R, NH), lambda i: (i, 0)),
          pl.BlockSpec((R, 1), lambda i: (i, 0)),
      ],
      out_shape=[
          jax.ShapeDtypeStruct((N, NH), jnp.float32),
          jax.ShapeDtypeStruct((N, 1), jnp.float32),
      ],
  )(deg_parts, x, W1)


def _tc2(p1, dis, b1, W2p, R=2000):
  _, N, NH = p1.shape
  WP = W2p.shape[1]

  def body(p_ref, dis_ref, b_ref, w_ref, t_ref):
    d = dis_ref[...]
    h = (p_ref[0] + p_ref[1]) * d + b_ref[...]
    h = jnp.maximum(h, 0.0)
    t_ref[...] = jnp.dot(h, w_ref[...],
                         preferred_element_type=jnp.float32) * d

  return pl.pallas_call(
      body,
      grid=(N // R,),
      in_specs=[
          pl.BlockSpec((_NC, R, NH), lambda i: (0, i, 0)),
          pl.BlockSpec((R, 1), lambda i: (i, 0)),
          pl.BlockSpec((1, NH), lambda i: (0, 0)),
          pl.BlockSpec((NH, WP), lambda i: (0, 0)),
      ],
      out_specs=pl.BlockSpec((R, WP), lambda i: (i, 0)),
      out_shape=jax.ShapeDtypeStruct((N, WP), jnp.float32),
  )(p1, dis, b1, W2p)


def _tc3(p2, dis, b2, R=2000):
  _, N, WP = p2.shape
  C = b2.shape[1]

  def body(p_ref, dis_ref, b_ref, o_ref):
    d = dis_ref[...]
    h = (p_ref[0] + p_ref[1])[:, :C] * d + b_ref[...]
    m = jnp.max(h, axis=1, keepdims=True)
    lse = m + jnp.log(jnp.sum(jnp.exp(h - m), axis=1, keepdims=True))
    o_ref[...] = h - lse

  return pl.pallas_call(
      body,
      grid=(N // R,),
      in_specs=[
          pl.BlockSpec((_NC, R, WP), lambda i: (0, i, 0)),
          pl.BlockSpec((R, 1), lambda i: (i, 0)),
          pl.BlockSpec((1, C), lambda i: (0, 0)),
      ],
      out_specs=pl.BlockSpec((R, C), lambda i: (i, 0)),
      out_shape=jax.ShapeDtypeStruct((N, C), jnp.float32),
  )(p2, dis, b2)


def kernel(x, edge_index, W1, b1, W2, b2):
  N, _ = x.shape
  NH = W1.shape[1]
  C = W2.shape[1]
  E = edge_index.shape[1]
  WP = 48
  W2p = jnp.pad(W2, ((0, 0), (0, WP - C)))
  NW = _NC * _NS
  ei_r = edge_index.reshape(2, NW, -1, _K2)

  deg_parts = _degree_kernel(N, E)(ei_r)
  t1, dis = _tc1(deg_parts, x, W1)
  p1 = _spmm_kernel(N, E, NH)(t1, ei_r)
  t2 = _tc2(p1, dis, b1.reshape(1, -1), W2p)
  p2 = _spmm_kernel(N, E, WP)(t2, ei_r)
  return _tc3(p2, dis, b2.reshape(1, -1))


# speedup vs baseline: 30.6293x; 1.0037x over previous
import functools

import jax
import jax.numpy as jnp
from jax import lax
from jax.experimental import pallas as pl
from jax.experimental.pallas import tpu as pltpu
from jax.experimental.pallas import tpu_sc as plsc

_NC = 2
_NS = 16
_K = 80
_DW = 16
_FB = 80


def _mesh():
  return plsc.VectorSubcoreMesh(core_axis_name="c", subcore_axis_name="s")


_K2 = 50


@functools.lru_cache(maxsize=None)
def _degree_kernel(N, E):
  NW = _NC * _NS
  EW = E // NW
  K2 = _K2
  NCH = EW // K2
  G = 8
  TB = N // _FB
  BPT = -(-TB // _NS)

  @functools.partial(
      pl.kernel,
      out_type=jax.ShapeDtypeStruct((_NC, N, _DW), jnp.float32),
      mesh=_mesh(),
      scratch_types=[
          pltpu.VMEM((NCH, K2), jnp.int32),
          pltpu.VMEM((K2, _DW), jnp.float32),
          pltpu.VMEM((_FB, _DW), jnp.float32),
          pltpu.VMEM_SHARED((N, _DW), jnp.float32),
          pltpu.SemaphoreType.DMA,
          pltpu.SemaphoreType.DMA,
      ],
      compiler_params=pltpu.CompilerParams(use_tc_tiling_on_sc=False),
  )
  def deg_k(ei_hbm, out_hbm, dst_all, ones_v, buf_v, acc_sh, isem, ssem):
    c = lax.axis_index("c")
    s = lax.axis_index("s")
    wid = c * _NS + s
    one = jnp.full((16,), 1.0, jnp.float32)
    zero = jnp.zeros((16,), jnp.float32)

    pltpu.async_copy(ei_hbm.at[1, wid], dst_all, isem)

    def fill(i, carry):
      ones_v[i, :] = one
      return carry

    lax.fori_loop(0, K2, fill, 0)

    def zrow(i, carry):
      buf_v[i, :] = zero
      return carry

    lax.fori_loop(0, _FB, zrow, 0)
    for j in range(BPT):
      blk = s + j * _NS

      @pl.when(blk < TB)
      def _():
        pltpu.sync_copy(buf_v, acc_sh.at[pl.ds(blk * _FB, _FB), :])

    pltpu.make_async_copy(ei_hbm.at[1, wid], dst_all, isem).wait()
    plsc.subcore_barrier()

    def body(g, carry):
      for j in range(G):
        pltpu.async_copy(ones_v, acc_sh.at[dst_all.at[g * G + j]], ssem,
                         add=True)
      for j in range(G):
        pltpu.make_async_copy(ones_v, acc_sh.at[dst_all.at[0]], ssem).wait()
      return carry

    lax.fori_loop(0, NCH // G, body, 0)
    plsc.subcore_barrier()
    for j in range(BPT):
      blk = s + j * _NS

      @pl.when(blk < TB)
      def _():
        r = blk * _FB
        pltpu.sync_copy(acc_sh.at[pl.ds(r, _FB), :], buf_v)
        pltpu.sync_copy(buf_v, out_hbm.at[c, pl.ds(r, _FB), :])

  return deg_k


@functools.lru_cache(maxsize=None)
def _spmm_kernel(N, E, W):
  NW = _NC * _NS
  EW = E // NW
  K2 = _K2
  NCH = EW // K2
  D = 5 if W > 64 else 8
  S = 3 if W > 64 else 5
  Q = NCH // D
  FBk = 40
  TB = N // FBk
  BPT = -(-TB // _NS)

  @functools.partial(
      pl.kernel,
      out_type=jax.ShapeDtypeStruct((_NC, N, W), jnp.float32),
      mesh=_mesh(),
      scratch_types=(
          [pltpu.VMEM((NCH, K2), jnp.int32)]
          + [pltpu.VMEM((K2,), jnp.int32) for _ in range(D)]
          + [pltpu.VMEM((K2, W), jnp.float32) for _ in range(D)]
          + [pltpu.VMEM_SHARED((N, W), jnp.float32)]
          + ([pltpu.VMEM_SHARED((N, W), jnp.float32)] if W <= 64 else [])
          + [pltpu.SemaphoreType.DMA for _ in range(3 * D + 1)]
      ),
      compiler_params=pltpu.CompilerParams(use_tc_tiling_on_sc=False),
  )
  def spmm_k(t_hbm, ei_hbm, out_hbm, *scr):
    dst_all = scr[0]
    srcb = scr[1:1 + D]
    rows = scr[1 + D:1 + 2 * D]
    acc_sh = scr[1 + 2 * D]
    off = 2 if W <= 64 else 1
    t_src = scr[1 + 2 * D + 1] if W <= 64 else t_hbm
    gs = scr[off + 1 + 2 * D:off + 1 + 3 * D]
    ss = scr[off + 1 + 3 * D:off + 1 + 4 * D]
    isems = scr[off + 1 + 4 * D:off + 1 + 5 * D]
    dsem = scr[off + 1 + 5 * D]
    c = lax.axis_index("c")
    s = lax.axis_index("s")
    wid = c * _NS + s
    zero = jnp.zeros((16,), jnp.float32)

    pltpu.async_copy(ei_hbm.at[1, wid], dst_all, dsem)

    r0 = rows[0]

    def zrow(i, carry):
      for j in range(W // 16):
        r0[i, pl.ds(j * 16, 16)] = zero
      return carry

    lax.fori_loop(0, FBk, zrow, 0)
    for j in range(BPT):
      blk = s + j * _NS

      @pl.when(blk < TB)
      def _():
        pltpu.sync_copy(r0.at[pl.ds(0, FBk), :],
                        acc_sh.at[pl.ds(blk * FBk, FBk), :])

    if W <= 64:
      RPT = N // _NS
      pltpu.sync_copy(t_hbm.at[pl.ds(s * RPT, RPT), :],
                      t_src.at[pl.ds(s * RPT, RPT), :])

    pltpu.make_async_copy(ei_hbm.at[1, wid], dst_all, dsem).wait()
    plsc.subcore_barrier()

    for k in range(S + 2):
      pltpu.async_copy(ei_hbm.at[0, wid, k], srcb[k % D], isems[k % D])
    for k in range(S):
      pltpu.make_async_copy(ei_hbm.at[0, wid, k], srcb[k % D],
                            isems[k % D]).wait()
      pltpu.async_copy(t_src.at[srcb[k % D]], rows[k % D], gs[k % D])

    def body(q, carry):
      for j in range(D):
        i = D * q + j
        bn = (j + S) % D
        bp = (j + S + 2) % D
        pltpu.make_async_copy(t_src.at[srcb[j]], rows[j], gs[j]).wait()

        def wait_prev_scatter(i=i, bn=bn):
          pltpu.make_async_copy(rows[bn], acc_sh.at[dst_all.at[i]],
                                ss[bn]).wait()

        if j >= D - S:
          wait_prev_scatter()
        else:
          pl.when(q > 0)(wait_prev_scatter)

        pltpu.async_copy(rows[j], acc_sh.at[dst_all.at[i]], ss[j], add=True)

        def start_next_gather(bn=bn):
          pltpu.make_async_copy(ei_hbm.at[0, wid, 0], srcb[bn],
                                isems[bn]).wait()
          pltpu.async_copy(t_src.at[srcb[bn]], rows[bn], gs[bn])

        if j < D - S:
          start_next_gather()
        else:
          pl.when(q < Q - 1)(start_next_gather)

        def prefetch_idx(i=i, bp=bp):
          pltpu.async_copy(ei_hbm.at[0, wid, i + S + 2], srcb[bp], isems[bp])

        if j < D - S - 2:
          prefetch_idx()
        else:
          pl.when(q < Q - 1)(prefetch_idx)
      return carry

    lax.fori_loop(0, Q, body, 0)
    for k in range(NCH - (D - S), NCH):
      pltpu.make_async_copy(rows[k % D], acc_sh.at[dst_all.at[NCH - 1]],
                            ss[k % D]).wait()
    plsc.subcore_barrier()
    for j in range(BPT):
      blk = s + j * _NS

      @pl.when(blk < TB)
      def _():
        r = blk * FBk
        pltpu.sync_copy(acc_sh.at[pl.ds(r, FBk), :], r0.at[pl.ds(0, FBk), :])
        pltpu.sync_copy(r0.at[pl.ds(0, FBk), :],
                        out_hbm.at[c, pl.ds(r, FBk), :])

  return spmm_k


def _tc1(deg_parts, x, W1, R=2000):
  N, NF = x.shape
  NH = W1.shape[1]

  def body(dp_ref, x_ref, w_ref, t_ref, dis_ref):
    deg = dp_ref[0][:, 0:1] + dp_ref[1][:, 0:1]
    dis = lax.rsqrt(jnp.maximum(deg, 1.0))
    s = jnp.dot(x_ref[...], w_ref[...], preferred_element_type=jnp.float32)
    t_ref[...] = s * dis
    dis_ref[...] = dis

  return pl.pallas_call(
      body,
      grid=(N // R,),
      in_specs=[
          pl.BlockSpec((_NC, R, _DW), lambda i: (0, i, 0)),
          pl.BlockSpec((R, NF), lambda i: (i, 0)),
          pl.BlockSpec((NF, NH), lambda i: (0, 0)),
      ],
      out_specs=[
          pl.BlockSpec((R, NH), lambda i: (i, 0)),
          pl.BlockSpec((R, 1), lambda i: (i, 0)),
      ],
      out_shape=[
          jax.ShapeDtypeStruct((N, NH), jnp.float32),
          jax.ShapeDtypeStruct((N, 1), jnp.float32),
      ],
  )(deg_parts, x, W1)


def _tc2(p1, dis, b1, W2p, R=2000):
  _, N, NH = p1.shape
  WP = W2p.shape[1]

  def body(p_ref, dis_ref, b_ref, w_ref, t_ref):
    d = dis_ref[...]
    h = (p_ref[0] + p_ref[1]) * d + b_ref[...]
    h = jnp.maximum(h, 0.0)
    t_ref[...] = jnp.dot(h, w_ref[...],
                         preferred_element_type=jnp.float32) * d

  return pl.pallas_call(
      body,
      grid=(N // R,),
      in_specs=[
          pl.BlockSpec((_NC, R, NH), lambda i: (0, i, 0)),
          pl.BlockSpec((R, 1), lambda i: (i, 0)),
          pl.BlockSpec((1, NH), lambda i: (0, 0)),
          pl.BlockSpec((NH, WP), lambda i: (0, 0)),
      ],
      out_specs=pl.BlockSpec((R, WP), lambda i: (i, 0)),
      out_shape=jax.ShapeDtypeStruct((N, WP), jnp.float32),
  )(p1, dis, b1, W2p)


def _tc3(p2, dis, b2, R=2000):
  _, N, WP = p2.shape
  C = b2.shape[1]

  def body(p_ref, dis_ref, b_ref, o_ref):
    d = dis_ref[...]
    h = (p_ref[0] + p_ref[1])[:, :C] * d + b_ref[...]
    m = jnp.max(h, axis=1, keepdims=True)
    lse = m + jnp.log(jnp.sum(jnp.exp(h - m), axis=1, keepdims=True))
    o_ref[...] = h - lse

  return pl.pallas_call(
      body,
      grid=(N // R,),
      in_specs=[
          pl.BlockSpec((_NC, R, WP), lambda i: (0, i, 0)),
          pl.BlockSpec((R, 1), lambda i: (i, 0)),
          pl.BlockSpec((1, C), lambda i: (0, 0)),
      ],
      out_specs=pl.BlockSpec((R, C), lambda i: (i, 0)),
      out_shape=jax.ShapeDtypeStruct((N, C), jnp.float32),
  )(p2, dis, b2)


def kernel(x, edge_index, W1, b1, W2, b2):
  N, _ = x.shape
  NH = W1.shape[1]
  C = W2.shape[1]
  E = edge_index.shape[1]
  WP = 48
  W2p = jnp.pad(W2, ((0, 0), (0, WP - C)))
  NW = _NC * _NS
  ei_r = edge_index.reshape(2, NW, -1, _K2)

  deg_parts = _degree_kernel(N, E)(ei_r)
  t1, dis = _tc1(deg_parts, x, W1)
  p1 = _spmm_kernel(N, E, NH)(t1, ei_r)
  t2 = _tc2(p1, dis, b1.reshape(1, -1), W2p)
  p2 = _spmm_kernel(N, E, WP)(t2, ei_r)
  return _tc3(p2, dis, b2.reshape(1, -1))


# speedup vs baseline: 31.4353x; 1.0263x over previous
import functools

import jax
import jax.numpy as jnp
from jax import lax
from jax.experimental import pallas as pl
from jax.experimental.pallas import tpu as pltpu
from jax.experimental.pallas import tpu_sc as plsc

_NC = 2
_NS = 16
_K = 80
_DW = 16
_FB = 80


def _mesh():
  return plsc.VectorSubcoreMesh(core_axis_name="c", subcore_axis_name="s")


_K2 = 50


@functools.lru_cache(maxsize=None)
def _degree_kernel(N, E):
  NW = _NC * _NS
  EW = E // NW
  K2 = _K2
  NCH = EW // K2
  G = 8
  TB = N // _FB
  BPT = -(-TB // _NS)

  @functools.partial(
      pl.kernel,
      out_type=jax.ShapeDtypeStruct((_NC, N, _DW), jnp.float32),
      mesh=_mesh(),
      scratch_types=[
          pltpu.VMEM((NCH, K2), jnp.int32),
          pltpu.VMEM((K2, _DW), jnp.float32),
          pltpu.VMEM((_FB, _DW), jnp.float32),
          pltpu.VMEM_SHARED((N, _DW), jnp.float32),
          pltpu.SemaphoreType.DMA,
          pltpu.SemaphoreType.DMA,
      ],
      compiler_params=pltpu.CompilerParams(use_tc_tiling_on_sc=False),
  )
  def deg_k(ei_hbm, out_hbm, dst_all, ones_v, buf_v, acc_sh, isem, ssem):
    c = lax.axis_index("c")
    s = lax.axis_index("s")
    wid = c * _NS + s
    one = jnp.full((16,), 1.0, jnp.float32)
    zero = jnp.zeros((16,), jnp.float32)

    pltpu.async_copy(ei_hbm.at[1, wid], dst_all, isem)

    def fill(i, carry):
      ones_v[i, :] = one
      return carry

    lax.fori_loop(0, K2, fill, 0)

    def zrow(i, carry):
      buf_v[i, :] = zero
      return carry

    lax.fori_loop(0, _FB, zrow, 0)
    for j in range(BPT):
      blk = s + j * _NS

      @pl.when(blk < TB)
      def _():
        pltpu.async_copy(buf_v, acc_sh.at[pl.ds(blk * _FB, _FB), :], ssem)

    for j in range(BPT):
      blk = s + j * _NS

      @pl.when(blk < TB)
      def _():
        pltpu.make_async_copy(buf_v, acc_sh.at[pl.ds(0, _FB), :], ssem).wait()

    pltpu.make_async_copy(ei_hbm.at[1, wid], dst_all, isem).wait()
    plsc.subcore_barrier()

    def body(g, carry):
      for j in range(G):
        pltpu.async_copy(ones_v, acc_sh.at[dst_all.at[g * G + j]], ssem,
                         add=True)
      for j in range(G):
        pltpu.make_async_copy(ones_v, acc_sh.at[dst_all.at[0]], ssem).wait()
      return carry

    lax.fori_loop(0, NCH // G, body, 0)
    plsc.subcore_barrier()
    RPT = N // _NS
    pltpu.sync_copy(acc_sh.at[pl.ds(s * RPT, RPT), :],
                    out_hbm.at[c, pl.ds(s * RPT, RPT), :])

  return deg_k


@functools.lru_cache(maxsize=None)
def _spmm_kernel(N, E, W):
  NW = _NC * _NS
  EW = E // NW
  K2 = _K2
  NCH = EW // K2
  D = 5 if W > 64 else 8
  S = 3 if W > 64 else 5
  Q = NCH // D
  FBk = 40
  TB = N // FBk
  BPT = -(-TB // _NS)

  @functools.partial(
      pl.kernel,
      out_type=jax.ShapeDtypeStruct((_NC, N, W), jnp.float32),
      mesh=_mesh(),
      scratch_types=(
          [pltpu.VMEM((NCH, K2), jnp.int32)]
          + [pltpu.VMEM((K2,), jnp.int32) for _ in range(D)]
          + [pltpu.VMEM((K2, W), jnp.float32) for _ in range(D)]
          + [pltpu.VMEM_SHARED((N, W), jnp.float32)]
          + ([pltpu.VMEM_SHARED((N, W), jnp.float32)] if W <= 64 else [])
          + [pltpu.SemaphoreType.DMA for _ in range(3 * D + 2)]
      ),
      compiler_params=pltpu.CompilerParams(use_tc_tiling_on_sc=False),
  )
  def spmm_k(t_hbm, ei_hbm, out_hbm, *scr):
    dst_all = scr[0]
    srcb = scr[1:1 + D]
    rows = scr[1 + D:1 + 2 * D]
    acc_sh = scr[1 + 2 * D]
    off = 2 if W <= 64 else 1
    t_src = scr[1 + 2 * D + 1] if W <= 64 else t_hbm
    gs = scr[off + 1 + 2 * D:off + 1 + 3 * D]
    ss = scr[off + 1 + 3 * D:off + 1 + 4 * D]
    isems = scr[off + 1 + 4 * D:off + 1 + 5 * D]
    dsem = scr[off + 1 + 5 * D]
    zsem = scr[off + 2 + 5 * D]
    c = lax.axis_index("c")
    s = lax.axis_index("s")
    wid = c * _NS + s
    zero = jnp.zeros((16,), jnp.float32)

    pltpu.async_copy(ei_hbm.at[1, wid], dst_all, dsem)

    r0 = rows[0]

    def zrow(i, carry):
      for j in range(W // 16):
        r0[i, pl.ds(j * 16, 16)] = zero
      return carry

    lax.fori_loop(0, FBk, zrow, 0)
    for j in range(BPT):
      blk = s + j * _NS

      @pl.when(blk < TB)
      def _():
        pltpu.async_copy(r0.at[pl.ds(0, FBk), :],
                         acc_sh.at[pl.ds(blk * FBk, FBk), :], zsem)

    for j in range(BPT):
      blk = s + j * _NS

      @pl.when(blk < TB)
      def _():
        pltpu.make_async_copy(r0.at[pl.ds(0, FBk), :],
                              acc_sh.at[pl.ds(0, FBk), :], zsem).wait()

    if W <= 64:
      RPT = N // _NS
      pltpu.sync_copy(t_hbm.at[pl.ds(s * RPT, RPT), :],
                      t_src.at[pl.ds(s * RPT, RPT), :])

    pltpu.make_async_copy(ei_hbm.at[1, wid], dst_all, dsem).wait()
    plsc.subcore_barrier()

    for k in range(S + 2):
      pltpu.async_copy(ei_hbm.at[0, wid, k], srcb[k % D], isems[k % D])
    for k in range(S):
      pltpu.make_async_copy(ei_hbm.at[0, wid, k], srcb[k % D],
                            isems[k % D]).wait()
      pltpu.async_copy(t_src.at[srcb[k % D]], rows[k % D], gs[k % D])

    def body(q, carry):
      for j in range(D):
        i = D * q + j
        bn = (j + S) % D
        bp = (j + S + 2) % D
        pltpu.make_async_copy(t_src.at[srcb[j]], rows[j], gs[j]).wait()

        def wait_prev_scatter(i=i, bn=bn):
          pltpu.make_async_copy(rows[bn], acc_sh.at[dst_all.at[i]],
                                ss[bn]).wait()

        if j >= D - S:
          wait_prev_scatter()
        else:
          pl.when(q > 0)(wait_prev_scatter)

        pltpu.async_copy(rows[j], acc_sh.at[dst_all.at[i]], ss[j], add=True)

        def start_next_gather(bn=bn):
          pltpu.make_async_copy(ei_hbm.at[0, wid, 0], srcb[bn],
                                isems[bn]).wait()
          pltpu.async_copy(t_src.at[srcb[bn]], rows[bn], gs[bn])

        if j < D - S:
          start_next_gather()
        else:
          pl.when(q < Q - 1)(start_next_gather)

        def prefetch_idx(i=i, bp=bp):
          pltpu.async_copy(ei_hbm.at[0, wid, i + S + 2], srcb[bp], isems[bp])

        if j < D - S - 2:
          prefetch_idx()
        else:
          pl.when(q < Q - 1)(prefetch_idx)
      return carry

    lax.fori_loop(0, Q, body, 0)
    for k in range(NCH - (D - S), NCH):
      pltpu.make_async_copy(rows[k % D], acc_sh.at[dst_all.at[NCH - 1]],
                            ss[k % D]).wait()
    plsc.subcore_barrier()
    RPT = N // _NS
    pltpu.sync_copy(acc_sh.at[pl.ds(s * RPT, RPT), :],
                    out_hbm.at[c, pl.ds(s * RPT, RPT), :])

  return spmm_k


def _tc1(deg_parts, x, W1, R=2000):
  N, NF = x.shape
  NH = W1.shape[1]

  def body(dp_ref, x_ref, w_ref, t_ref, dis_ref):
    deg = dp_ref[0][:, 0:1] + dp_ref[1][:, 0:1]
    dis = lax.rsqrt(jnp.maximum(deg, 1.0))
    s = jnp.dot(x_ref[...], w_ref[...], preferred_element_type=jnp.float32)
    t_ref[...] = s * dis
    dis_ref[...] = dis

  return pl.pallas_call(
      body,
      grid=(N // R,),
      in_specs=[
          pl.BlockSpec((_NC, R, _DW), lambda i: (0, i, 0)),
          pl.BlockSpec((R, NF), lambda i: (i, 0)),
          pl.BlockSpec((NF, NH), lambda i: (0, 0)),
      ],
      out_specs=[
          pl.BlockSpec((R, NH), lambda i: (i, 0)),
          pl.BlockSpec((R, 1), lambda i: (i, 0)),
      ],
      out_shape=[
          jax.ShapeDtypeStruct((N, NH), jnp.float32),
          jax.ShapeDtypeStruct((N, 1), jnp.float32),
      ],
  )(deg_parts, x, W1)


def _tc2(p1, dis, b1, W2p, R=2000):
  _, N, NH = p1.shape
  WP = W2p.shape[1]

  def body(p_ref, dis_ref, b_ref, w_ref, t_ref):
    d = dis_ref[...]
    h = (p_ref[0] + p_ref[1]) * d + b_ref[...]
    h = jnp.maximum(h, 0.0)
    t_ref[...] = jnp.dot(h, w_ref[...],
                         preferred_element_type=jnp.float32) * d

  return pl.pallas_call(
      body,
      grid=(N // R,),
      in_specs=[
          pl.BlockSpec((_NC, R, NH), lambda i: (0, i, 0)),
          pl.BlockSpec((R, 1), lambda i: (i, 0)),
          pl.BlockSpec((1, NH), lambda i: (0, 0)),
          pl.BlockSpec((NH, WP), lambda i: (0, 0)),
      ],
      out_specs=pl.BlockSpec((R, WP), lambda i: (i, 0)),
      out_shape=jax.ShapeDtypeStruct((N, WP), jnp.float32),
  )(p1, dis, b1, W2p)


def _tc3(p2, dis, b2, R=2000):
  _, N, WP = p2.shape
  C = b2.shape[1]

  def body(p_ref, dis_ref, b_ref, o_ref):
    d = dis_ref[...]
    h = (p_ref[0] + p_ref[1])[:, :C] * d + b_ref[...]
    m = jnp.max(h, axis=1, keepdims=True)
    lse = m + jnp.log(jnp.sum(jnp.exp(h - m), axis=1, keepdims=True))
    o_ref[...] = h - lse

  return pl.pallas_call(
      body,
      grid=(N // R,),
      in_specs=[
          pl.BlockSpec((_NC, R, WP), lambda i: (0, i, 0)),
          pl.BlockSpec((R, 1), lambda i: (i, 0)),
          pl.BlockSpec((1, C), lambda i: (0, 0)),
      ],
      out_specs=pl.BlockSpec((R, C), lambda i: (i, 0)),
      out_shape=jax.ShapeDtypeStruct((N, C), jnp.float32),
  )(p2, dis, b2)


def kernel(x, edge_index, W1, b1, W2, b2):
  N, _ = x.shape
  NH = W1.shape[1]
  C = W2.shape[1]
  E = edge_index.shape[1]
  WP = 48
  W2p = jnp.pad(W2, ((0, 0), (0, WP - C)))
  NW = _NC * _NS
  ei_r = edge_index.reshape(2, NW, -1, _K2)

  deg_parts = _degree_kernel(N, E)(ei_r)
  t1, dis = _tc1(deg_parts, x, W1)
  p1 = _spmm_kernel(N, E, NH)(t1, ei_r)
  t2 = _tc2(p1, dis, b1.reshape(1, -1), W2p)
  p2 = _spmm_kernel(N, E, WP)(t2, ei_r)
  return _tc3(p2, dis, b2.reshape(1, -1))


# speedup vs baseline: 31.4504x; 1.0005x over previous
import functools

import jax
import jax.numpy as jnp
from jax import lax
from jax.experimental import pallas as pl
from jax.experimental.pallas import tpu as pltpu
from jax.experimental.pallas import tpu_sc as plsc

_NC = 2
_NS = 16
_K = 80
_DW = 16
_FB = 80


def _mesh():
  return plsc.VectorSubcoreMesh(core_axis_name="c", subcore_axis_name="s")


_K2 = 50


@functools.lru_cache(maxsize=None)
def _degree_kernel(N, E):
  NW = _NC * _NS
  EW = E // NW
  K2 = _K2
  NCH = EW // K2
  G = 10
  TB = N // _FB
  BPT = -(-TB // _NS)

  @functools.partial(
      pl.kernel,
      out_type=jax.ShapeDtypeStruct((_NC, N, _DW), jnp.float32),
      mesh=_mesh(),
      scratch_types=[
          pltpu.VMEM((NCH, K2), jnp.int32),
          pltpu.VMEM((K2, _DW), jnp.float32),
          pltpu.VMEM((_FB, _DW), jnp.float32),
          pltpu.VMEM_SHARED((N, _DW), jnp.float32),
          pltpu.SemaphoreType.DMA,
          pltpu.SemaphoreType.DMA,
      ],
      compiler_params=pltpu.CompilerParams(use_tc_tiling_on_sc=False),
  )
  def deg_k(ei_hbm, out_hbm, dst_all, ones_v, buf_v, acc_sh, isem, ssem):
    c = lax.axis_index("c")
    s = lax.axis_index("s")
    wid = c * _NS + s
    one = jnp.full((16,), 1.0, jnp.float32)
    zero = jnp.zeros((16,), jnp.float32)

    pltpu.async_copy(ei_hbm.at[1, wid], dst_all, isem)

    def fill(i, carry):
      ones_v[i, :] = one
      return carry

    lax.fori_loop(0, K2, fill, 0)

    def zrow(i, carry):
      buf_v[i, :] = zero
      return carry

    lax.fori_loop(0, _FB, zrow, 0)
    for j in range(BPT):
      blk = s + j * _NS

      @pl.when(blk < TB)
      def _():
        pltpu.async_copy(buf_v, acc_sh.at[pl.ds(blk * _FB, _FB), :], ssem)

    for j in range(BPT):
      blk = s + j * _NS

      @pl.when(blk < TB)
      def _():
        pltpu.make_async_copy(buf_v, acc_sh.at[pl.ds(0, _FB), :], ssem).wait()

    pltpu.make_async_copy(ei_hbm.at[1, wid], dst_all, isem).wait()
    plsc.subcore_barrier()

    def body(g, carry):
      for j in range(G):
        pltpu.async_copy(ones_v, acc_sh.at[dst_all.at[g * G + j]], ssem,
                         add=True)
      for j in range(G):
        pltpu.make_async_copy(ones_v, acc_sh.at[dst_all.at[0]], ssem).wait()
      return carry

    lax.fori_loop(0, NCH // G, body, 0)
    plsc.subcore_barrier()
    RPT = N // _NS
    pltpu.sync_copy(acc_sh.at[pl.ds(s * RPT, RPT), :],
                    out_hbm.at[c, pl.ds(s * RPT, RPT), :])

  return deg_k


@functools.lru_cache(maxsize=None)
def _spmm_kernel(N, E, W):
  NW = _NC * _NS
  EW = E // NW
  K2 = _K2
  NCH = EW // K2
  D = 5 if W > 64 else 8
  S = 3 if W > 64 else 6
  Q = NCH // D
  FBk = 40
  TB = N // FBk
  BPT = -(-TB // _NS)

  @functools.partial(
      pl.kernel,
      out_type=jax.ShapeDtypeStruct((_NC, N, W), jnp.float32),
      mesh=_mesh(),
      scratch_types=(
          [pltpu.VMEM((NCH, K2), jnp.int32)]
          + [pltpu.VMEM((K2,), jnp.int32) for _ in range(D)]
          + [pltpu.VMEM((K2, W), jnp.float32) for _ in range(D)]
          + [pltpu.VMEM_SHARED((N, W), jnp.float32)]
          + ([pltpu.VMEM_SHARED((N, W), jnp.float32)] if W <= 64 else [])
          + [pltpu.SemaphoreType.DMA for _ in range(3 * D + 2)]
      ),
      compiler_params=pltpu.CompilerParams(use_tc_tiling_on_sc=False),
  )
  def spmm_k(t_hbm, ei_hbm, out_hbm, *scr):
    dst_all = scr[0]
    srcb = scr[1:1 + D]
    rows = scr[1 + D:1 + 2 * D]
    acc_sh = scr[1 + 2 * D]
    off = 2 if W <= 64 else 1
    t_src = scr[1 + 2 * D + 1] if W <= 64 else t_hbm
    gs = scr[off + 1 + 2 * D:off + 1 + 3 * D]
    ss = scr[off + 1 + 3 * D:off + 1 + 4 * D]
    isems = scr[off + 1 + 4 * D:off + 1 + 5 * D]
    dsem = scr[off + 1 + 5 * D]
    zsem = scr[off + 2 + 5 * D]
    c = lax.axis_index("c")
    s = lax.axis_index("s")
    wid = c * _NS + s
    zero = jnp.zeros((16,), jnp.float32)

    pltpu.async_copy(ei_hbm.at[1, wid], dst_all, dsem)

    r0 = rows[0]

    def zrow(i, carry):
      for j in range(W // 16):
        r0[i, pl.ds(j * 16, 16)] = zero
      return carry

    lax.fori_loop(0, FBk, zrow, 0)
    for j in range(BPT):
      blk = s + j * _NS

      @pl.when(blk < TB)
      def _():
        pltpu.async_copy(r0.at[pl.ds(0, FBk), :],
                         acc_sh.at[pl.ds(blk * FBk, FBk), :], zsem)

    for j in range(BPT):
      blk = s + j * _NS

      @pl.when(blk < TB)
      def _():
        pltpu.make_async_copy(r0.at[pl.ds(0, FBk), :],
                              acc_sh.at[pl.ds(0, FBk), :], zsem).wait()

    if W <= 64:
      RPT = N // _NS
      pltpu.sync_copy(t_hbm.at[pl.ds(s * RPT, RPT), :],
                      t_src.at[pl.ds(s * RPT, RPT), :])

    pltpu.make_async_copy(ei_hbm.at[1, wid], dst_all, dsem).wait()
    plsc.subcore_barrier()

    for k in range(S + 2):
      pltpu.async_copy(ei_hbm.at[0, wid, k], srcb[k % D], isems[k % D])
    for k in range(S):
      pltpu.make_async_copy(ei_hbm.at[0, wid, k], srcb[k % D],
                            isems[k % D]).wait()
      pltpu.async_copy(t_src.at[srcb[k % D]], rows[k % D], gs[k % D])

    def body(q, carry):
      for j in range(D):
        i = D * q + j
        bn = (j + S) % D
        bp = (j + S + 2) % D
        pltpu.make_async_copy(t_src.at[srcb[j]], rows[j], gs[j]).wait()

        def wait_prev_scatter(i=i, bn=bn):
          pltpu.make_async_copy(rows[bn], acc_sh.at[dst_all.at[i]],
                                ss[bn]).wait()

        if j >= D - S:
          wait_prev_scatter()
        else:
          pl.when(q > 0)(wait_prev_scatter)

        pltpu.async_copy(rows[j], acc_sh.at[dst_all.at[i]], ss[j], add=True)

        def start_next_gather(bn=bn):
          pltpu.make_async_copy(ei_hbm.at[0, wid, 0], srcb[bn],
                                isems[bn]).wait()
          pltpu.async_copy(t_src.at[srcb[bn]], rows[bn], gs[bn])

        if j < D - S:
          start_next_gather()
        else:
          pl.when(q < Q - 1)(start_next_gather)

        def prefetch_idx(i=i, bp=bp):
          pltpu.async_copy(ei_hbm.at[0, wid, i + S + 2], srcb[bp], isems[bp])

        if j < D - S - 2:
          prefetch_idx()
        else:
          pl.when(q < Q - 1)(prefetch_idx)
      return carry

    lax.fori_loop(0, Q, body, 0)
    for k in range(NCH - (D - S), NCH):
      pltpu.make_async_copy(rows[k % D], acc_sh.at[dst_all.at[NCH - 1]],
                            ss[k % D]).wait()
    plsc.subcore_barrier()
    RPT = N // _NS
    pltpu.sync_copy(acc_sh.at[pl.ds(s * RPT, RPT), :],
                    out_hbm.at[c, pl.ds(s * RPT, RPT), :])

  return spmm_k


def _tc1(deg_parts, x, W1, R=2000):
  N, NF = x.shape
  NH = W1.shape[1]

  def body(dp_ref, x_ref, w_ref, t_ref, dis_ref):
    deg = dp_ref[0][:, 0:1] + dp_ref[1][:, 0:1]
    dis = lax.rsqrt(jnp.maximum(deg, 1.0))
    s = jnp.dot(x_ref[...], w_ref[...], preferred_element_type=jnp.float32)
    t_ref[...] = s * dis
    dis_ref[...] = dis

  return pl.pallas_call(
      body,
      grid=(N // R,),
      in_specs=[
          pl.BlockSpec((_NC, R, _DW), lambda i: (0, i, 0)),
          pl.BlockSpec((R, NF), lambda i: (i, 0)),
          pl.BlockSpec((NF, NH), lambda i: (0, 0)),
      ],
      out_specs=[
          pl.BlockSpec((R, NH), lambda i: (i, 0)),
          pl.BlockSpec((R, 1), lambda i: (i, 0)),
      ],
      out_shape=[
          jax.ShapeDtypeStruct((N, NH), jnp.float32),
          jax.ShapeDtypeStruct((N, 1), jnp.float32),
      ],
  )(deg_parts, x, W1)


def _tc2(p1, dis, b1, W2p, R=2000):
  _, N, NH = p1.shape
  WP = W2p.shape[1]

  def body(p_ref, dis_ref, b_ref, w_ref, t_ref):
    d = dis_ref[...]
    h = (p_ref[0] + p_ref[1]) * d + b_ref[...]
    h = jnp.maximum(h, 0.0)
    t_ref[...] = jnp.dot(h, w_ref[...],
                         preferred_element_type=jnp.float32) * d

  return pl.pallas_call(
      body,
      grid=(N // R,),
      in_specs=[
          pl.BlockSpec((_NC, R, NH), lambda i: (0, i, 0)),
          pl.BlockSpec((R, 1), lambda i: (i, 0)),
          pl.BlockSpec((1, NH), lambda i: (0, 0)),
          pl.BlockSpec((NH, WP), lambda i: (0, 0)),
      ],
      out_specs=pl.BlockSpec((R, WP), lambda i: (i, 0)),
      out_shape=jax.ShapeDtypeStruct((N, WP), jnp.float32),
  )(p1, dis, b1, W2p)


def _tc3(p2, dis, b2, R=2000):
  _, N, WP = p2.shape
  C = b2.shape[1]

  def body(p_ref, dis_ref, b_ref, o_ref):
    d = dis_ref[...]
    h = (p_ref[0] + p_ref[1])[:, :C] * d + b_ref[...]
    m = jnp.max(h, axis=1, keepdims=True)
    lse = m + jnp.log(jnp.sum(jnp.exp(h - m), axis=1, keepdims=True))
    o_ref[...] = h - lse

  return pl.pallas_call(
      body,
      grid=(N // R,),
      in_specs=[
          pl.BlockSpec((_NC, R, WP), lambda i: (0, i, 0)),
          pl.BlockSpec((R, 1), lambda i: (i, 0)),
          pl.BlockSpec((1, C), lambda i: (0, 0)),
      ],
      out_specs=pl.BlockSpec((R, C), lambda i: (i, 0)),
      out_shape=jax.ShapeDtypeStruct((N, C), jnp.float32),
  )(p2, dis, b2)


def kernel(x, edge_index, W1, b1, W2, b2):
  N, _ = x.shape
  NH = W1.shape[1]
  C = W2.shape[1]
  E = edge_index.shape[1]
  WP = 48
  W2p = jnp.pad(W2, ((0, 0), (0, WP - C)))
  NW = _NC * _NS
  ei_r = edge_index.reshape(2, NW, -1, _K2)

  deg_parts = _degree_kernel(N, E)(ei_r)
  t1, dis = _tc1(deg_parts, x, W1)
  p1 = _spmm_kernel(N, E, NH)(t1, ei_r)
  t2 = _tc2(p1, dis, b1.reshape(1, -1), W2p)
  p2 = _spmm_kernel(N, E, WP)(t2, ei_r)
  return _tc3(p2, dis, b2.reshape(1, -1))
